# Initial kernel scaffold; baseline (speedup 1.0000x reference)
import jax
import jax.numpy as jnp
from jax.experimental import pallas as pl


def kernel(x, edge_index, ts_beats, divs_pq, onset_div, duration_div, not_removed_notes, computation_notes, target, params_op, params_idx, params_pitch):
    raise NotImplementedError("write your pallas kernel here")


# speedup vs baseline: 24.4619x; 24.4619x over previous
import functools

import jax
import jax.numpy as jnp
from jax import lax
from jax.experimental import pallas as pl
from jax.experimental.pallas import tpu as pltpu
from jax.experimental.pallas import tpu_sc as plsc

N = 10000
D = 128
E = 320000
L = 16
NC = 2
NS = 16
NW = NC * NS
EPW = E // NW
CHUNKS = 80
CLEN = EPW // CHUNKS
RPT = 624
REM = N - NS * RPT
RBLK = 2000
GRID = N // RBLK
NCOMP = 1000
NCOMP_PAD = 1024

f32 = jnp.float32
i32 = jnp.int32


def _proj_body(x_ref, pt_ref, bp_ref, w1t_ref, wrt_ref, y1_ref, xr_ref):
    xb = x_ref[...]
    p = jnp.maximum(jnp.dot(xb, pt_ref[...], preferred_element_type=f32) + bp_ref[...], 0.0)
    y1 = jnp.dot(p, w1t_ref[...], preferred_element_type=f32)
    lane = lax.broadcasted_iota(i32, (RBLK, L), 1)
    y1_ref[...] = y1 + jnp.where(lane == 14, 1.0, 0.0)
    xr_ref[...] = jnp.dot(xb, wrt_ref[...], preferred_element_type=f32)


def _proj(x, pt, bp, w1t, wrt):
    return pl.pallas_call(
        _proj_body,
        grid=(GRID,),
        in_specs=[
            pl.BlockSpec((RBLK, D), lambda i: (i, 0)),
            pl.BlockSpec((D, 2 * D), lambda i: (0, 0)),
            pl.BlockSpec((1, 2 * D), lambda i: (0, 0)),
            pl.BlockSpec((2 * D, L), lambda i: (0, 0)),
            pl.BlockSpec((D, L), lambda i: (0, 0)),
        ],
        out_specs=[
            pl.BlockSpec((RBLK, L), lambda i: (i, 0)),
            pl.BlockSpec((RBLK, L), lambda i: (i, 0)),
        ],
        out_shape=[
            jax.ShapeDtypeStruct((N, L), f32),
            jax.ShapeDtypeStruct((N, L), f32),
        ],
    )(x, pt, bp, w1t, wrt)


def _edge_agg_body(y_hbm, src_hbm, dst_hbm, zer_hbm, out_hbm,
                   src_v, dst_v, rows_a, rows_b, shared, sem_a, sem_b):
    c = lax.axis_index("c")
    s = lax.axis_index("s")
    w = c * NS + s
    pltpu.sync_copy(zer_hbm.at[pl.ds(s * RPT, RPT)], shared.at[pl.ds(s * RPT, RPT)])

    @pl.when(s == NS - 1)
    def _():
        pltpu.sync_copy(zer_hbm.at[pl.ds(NS * RPT, REM)], shared.at[pl.ds(NS * RPT, REM)])
    pltpu.sync_copy(src_hbm.at[w], src_v)
    pltpu.sync_copy(dst_hbm.at[w], dst_v)
    plsc.subcore_barrier()
    cp0 = pltpu.async_copy(y_hbm.at[src_v.at[0]], rows_a, sem_a)

    def body(j, carry):
        del carry
        even = j % 2 == 0

        @pl.when(even)
        def _():
            pltpu.make_async_copy(y_hbm.at[src_v.at[j]], rows_a, sem_a).wait()

            @pl.when(j + 1 < CHUNKS)
            def _():
                pltpu.async_copy(y_hbm.at[src_v.at[j + 1]], rows_b, sem_b)
            pltpu.sync_copy(rows_a, shared.at[dst_v.at[j]], add=True)

        @pl.when(jnp.logical_not(even))
        def _():
            pltpu.make_async_copy(y_hbm.at[src_v.at[j]], rows_b, sem_b).wait()

            @pl.when(j + 1 < CHUNKS)
            def _():
                pltpu.async_copy(y_hbm.at[src_v.at[j + 1]], rows_a, sem_a)
            pltpu.sync_copy(rows_b, shared.at[dst_v.at[j]], add=True)
        return 0

    del cp0
    lax.fori_loop(0, CHUNKS, body, 0)
    plsc.subcore_barrier()
    pltpu.sync_copy(shared.at[pl.ds(s * RPT, RPT)], out_hbm.at[c, pl.ds(s * RPT, RPT)])

    @pl.when(s == NS - 1)
    def _():
        pltpu.sync_copy(shared.at[pl.ds(NS * RPT, REM)], out_hbm.at[c, pl.ds(NS * RPT, REM)])


def _edge_agg(y, src3, dst3, zer):
    k = pl.kernel(
        _edge_agg_body,
        out_type=jax.ShapeDtypeStruct((NC, N, L), f32),
        mesh=plsc.VectorSubcoreMesh(core_axis_name="c", subcore_axis_name="s"),
        scratch_types=[
            pltpu.VMEM((CHUNKS, CLEN), i32),
            pltpu.VMEM((CHUNKS, CLEN), i32),
            pltpu.VMEM((CLEN, L), f32),
            pltpu.VMEM((CLEN, L), f32),
            pltpu.VMEM_SHARED((N, L), f32),
            pltpu.SemaphoreType.DMA,
            pltpu.SemaphoreType.DMA,
        ],
        compiler_params=pltpu.CompilerParams(use_tc_tiling_on_sc=False),
    )
    return k(y, src3, dst3, zer)


def _mid_body(parts_ref, xr_ref, w2t_ref, wr2t_ref, blg_ref, y2_ref, xr2_ref):
    a = parts_ref[0] + parts_ref[1]
    blcat = blg_ref[0:1, :]
    gcat = blg_ref[1:2, :]
    bcat = blg_ref[2:3, :]
    lane = lax.broadcasted_iota(i32, (RBLK, L), 1)
    m_idx = lane < 2
    m_pitch = (lane >= 2) & (lane < 14)
    cnt = jnp.sum(jnp.where(lane == 14, a, 0.0), axis=1, keepdims=True)
    inv = 1.0 / jnp.maximum(cnt, 1.0)
    o = jnp.where(lane < 14, a * inv + blcat + xr_ref[...], 0.0)
    ssq_i = jnp.sum(jnp.where(m_idx, o * o, 0.0), axis=1, keepdims=True)
    ssq_p = jnp.sum(jnp.where(m_pitch, o * o, 0.0), axis=1, keepdims=True)
    nrm = jnp.sqrt(jnp.where(m_idx, ssq_i, ssq_p))
    o = o / jnp.maximum(nrm, 1e-12)
    o = jnp.maximum(o, 0.0)
    mean = jnp.where(m_idx,
                     jnp.sum(jnp.where(m_idx, o, 0.0), 1, keepdims=True) * (1.0 / 2.0),
                     jnp.sum(jnp.where(m_pitch, o, 0.0), 1, keepdims=True) * (1.0 / 12.0))
    dlt = o - mean
    var = jnp.where(m_idx,
                    jnp.sum(jnp.where(m_idx, dlt * dlt, 0.0), 1, keepdims=True) * (1.0 / 2.0),
                    jnp.sum(jnp.where(m_pitch, dlt * dlt, 0.0), 1, keepdims=True) * (1.0 / 12.0))
    h1 = jnp.where(lane < 14, dlt / jnp.sqrt(var + 1e-5) * gcat + bcat, 0.0)
    y2_ref[...] = jnp.dot(h1, w2t_ref[...], preferred_element_type=f32)
    xr2_ref[...] = (jnp.dot(h1, wr2t_ref[...], preferred_element_type=f32)
                    + jnp.where(lane == 14, inv, 0.0))


def _mid(parts, xr, w2t, wr2t, blg):
    return pl.pallas_call(
        _mid_body,
        grid=(GRID,),
        in_specs=[
            pl.BlockSpec((NC, RBLK, L), lambda i: (0, i, 0)),
            pl.BlockSpec((RBLK, L), lambda i: (i, 0)),
            pl.BlockSpec((L, L), lambda i: (0, 0)),
            pl.BlockSpec((L, L), lambda i: (0, 0)),
            pl.BlockSpec((3, L), lambda i: (0, 0)),
        ],
        out_specs=[
            pl.BlockSpec((RBLK, L), lambda i: (i, 0)),
            pl.BlockSpec((RBLK, L), lambda i: (i, 0)),
        ],
        out_shape=[
            jax.ShapeDtypeStruct((N, L), f32),
            jax.ShapeDtypeStruct((N, L), f32),
        ],
    )(parts, xr, w2t, wr2t, blg)


def _final_body(parts_ref, xr2_ref, bl2_ref, emb_ref):
    a = parts_ref[0] + parts_ref[1]
    xr2 = xr2_ref[...]
    lane = lax.broadcasted_iota(i32, (RBLK, L), 1)
    inv = jnp.sum(jnp.where(lane == 14, xr2, 0.0), axis=1, keepdims=True)
    e = jnp.where(lane < 14, a * inv + bl2_ref[...] + xr2, 0.0)
    scores = jnp.sum(jnp.where(lane < 2, e, 0.0), axis=1, keepdims=True)
    emb_ref[...] = e + jnp.where(lane == 15, scores, 0.0)


def _final(parts, xr2, bl2):
    return pl.pallas_call(
        _final_body,
        grid=(GRID,),
        in_specs=[
            pl.BlockSpec((NC, RBLK, L), lambda i: (0, i, 0)),
            pl.BlockSpec((RBLK, L), lambda i: (i, 0)),
            pl.BlockSpec((1, L), lambda i: (0, 0)),
        ],
        out_specs=pl.BlockSpec((RBLK, L), lambda i: (i, 0)),
        out_shape=jax.ShapeDtypeStruct((N, L), f32),
    )(parts, xr2, bl2)


def _select_body(emb_hbm, sidx_hbm, cid_hbm, out_hbm,
                 sidx_v, cid_v, sc_v, row_v, out_v, sem):
    c = lax.axis_index("c")
    s = lax.axis_index("s")

    @pl.when((c == 0) & (s == 0))
    def _():
        pltpu.sync_copy(sidx_hbm, sidx_v)
        pltpu.sync_copy(cid_hbm, cid_v)
        best_v = jnp.full((L,), -3e38, f32)
        best_n = jnp.zeros((L,), i32)
        best_p = jnp.full((L,), 2**30, i32)
        lane = lax.iota(i32, L)
        for j in range(NCOMP_PAD // 128):
            pltpu.async_copy(emb_hbm.at[sidx_v.at[j]], sc_v, sem).wait()
            for k in range(128 // L):
                v = sc_v[pl.ds(k * L, L)]
                cid = cid_v.at[j][pl.ds(k * L, L)]
                pos = lane + (j * 128 + k * L)
                upd = (v > best_v) | ((v == best_v) & (pos < best_p))
                best_v = jnp.where(upd, v, best_v)
                best_n = jnp.where(upd, cid, best_n)
                best_p = jnp.where(upd, pos, best_p)
        bv, bn, bp = best_v[0], best_n[0], best_p[0]
        for l in range(1, L):
            v = best_v[l]
            take = (v > bv) | ((v == bv) & (best_p[l] < bp))
            bv = jnp.where(take, v, bv)
            bn = jnp.where(take, best_n[l], bn)
            bp = jnp.where(take, best_p[l], bp)
        pltpu.sync_copy(emb_hbm.at[pl.ds(bn * L, L)], row_v)
        rv = row_v[pl.ds(0, L)]
        pv = rv[2]
        pi = jnp.int32(0)
        for l in range(3, 14):
            v = rv[l]
            take = v > pv
            pv = jnp.where(take, v, pv)
            pi = jnp.where(take, jnp.int32(l - 2), pi)
        out_v[...] = jnp.where(lane == 0, bn, 0) + jnp.where(lane == 1, pi, 0)
        pltpu.sync_copy(out_v, out_hbm)


def _select(emb_flat, sidx, cid):
    k = pl.kernel(
        _select_body,
        out_type=jax.ShapeDtypeStruct((L,), i32),
        mesh=plsc.VectorSubcoreMesh(core_axis_name="c", subcore_axis_name="s"),
        scratch_types=[
            pltpu.VMEM((NCOMP_PAD // 128, 128), i32),
            pltpu.VMEM((NCOMP_PAD // 128, 128), i32),
            pltpu.VMEM((128,), f32),
            pltpu.VMEM((L,), f32),
            pltpu.VMEM((L,), i32),
            pltpu.SemaphoreType.DMA,
        ],
    )
    return k(emb_flat, sidx, cid)


def kernel(x, edge_index, ts_beats, divs_pq, onset_div, duration_div,
           not_removed_notes, computation_notes, target,
           params_op, params_idx, params_pitch):
    del ts_beats, divs_pq, onset_div, duration_div, not_removed_notes
    del target, params_op
    pi, pp = params_idx, params_pitch

    pt = jnp.concatenate([pi['c1']['Wp'], pp['c1']['Wp']], axis=0).T
    bp = jnp.concatenate([pi['c1']['bp'], pp['c1']['bp']]).reshape(1, 2 * D)
    w1t = (jnp.zeros((2 * D, L), f32)
           .at[:D, 0:2].set(pi['c1']['Wl'].T)
           .at[D:, 2:14].set(pp['c1']['Wl'].T))
    wrt = (jnp.zeros((D, L), f32)
           .at[:, 0:2].set(pi['c1']['Wr'].T)
           .at[:, 2:14].set(pp['c1']['Wr'].T))
    blg = (jnp.zeros((3, L), f32)
           .at[0, 0:2].set(pi['c1']['bl']).at[0, 2:14].set(pp['c1']['bl'])
           .at[1, 0:2].set(pi['ln_g']).at[1, 2:14].set(pp['ln_g'])
           .at[2, 0:2].set(pi['ln_b']).at[2, 2:14].set(pp['ln_b']))
    w2t = (jnp.zeros((L, L), f32)
           .at[0:2, 0:2].set(pi['c2']['Wl'].T)
           .at[2:14, 2:14].set(pp['c2']['Wl'].T))
    wr2t = (jnp.zeros((L, L), f32)
            .at[0:2, 0:2].set(pi['c2']['Wr'].T)
            .at[2:14, 2:14].set(pp['c2']['Wr'].T))
    bl2 = (jnp.zeros((1, L), f32)
           .at[0, 0:2].set(pi['c2']['bl']).at[0, 2:14].set(pp['c2']['bl']))

    src3 = edge_index[0].astype(i32).reshape(NW, CHUNKS, CLEN)
    dst3 = edge_index[1].astype(i32).reshape(NW, CHUNKS, CLEN)
    zer = jnp.zeros((N, L), f32)

    comp = jnp.sort(computation_notes).astype(i32)
    comp_pad = jnp.concatenate([comp, jnp.broadcast_to(comp[0], (NCOMP_PAD - NCOMP,))])
    cid = comp_pad.reshape(NCOMP_PAD // 128, 128)
    sidx = cid * L + 15

    y1, xr = _proj(x, pt, bp, w1t, wrt)
    parts1 = _edge_agg(y1, src3, dst3, zer)
    y2, xr2 = _mid(parts1, xr, w2t, wr2t, blg)
    parts2 = _edge_agg(y2, src3, dst3, zer)
    emb = _final(parts2, xr2, bl2)
    sel = _select(emb.reshape(N * L), sidx, cid)

    note_index = sel[0]
    new_pitch = sel[1]
    return x.at[note_index, :12].set(jax.nn.one_hot(new_pitch, 12, dtype=x.dtype))


# speedup vs baseline: 36.2886x; 1.4835x over previous
import functools

import jax
import jax.numpy as jnp
from jax import lax
from jax.experimental import pallas as pl
from jax.experimental.pallas import tpu as pltpu
from jax.experimental.pallas import tpu_sc as plsc

N = 10000
D = 128
E = 320000
L = 16
NC = 2
NS = 16
NW = NC * NS
EPW = E // NW
CHUNKS = 80
CLEN = EPW // CHUNKS
RPT = 624
REM = N - NS * RPT
RBLK = 2000
GRID = N // RBLK
NCOMP = 1000
NCOMP_PAD = 1024

f32 = jnp.float32
i32 = jnp.int32


def _proj_body(x_ref, pt_ref, bp_ref, w1t_ref, wrt_ref, y1_ref, xr_ref):
    xb = x_ref[...]
    p = jnp.maximum(jnp.dot(xb, pt_ref[...], preferred_element_type=f32) + bp_ref[...], 0.0)
    y1 = jnp.dot(p, w1t_ref[...], preferred_element_type=f32)
    lane = lax.broadcasted_iota(i32, (RBLK, L), 1)
    y1_ref[...] = y1 + jnp.where(lane == 14, 1.0, 0.0)
    xr_ref[...] = jnp.dot(xb, wrt_ref[...], preferred_element_type=f32)


def _proj(x, pt, bp, w1t, wrt):
    return pl.pallas_call(
        _proj_body,
        grid=(GRID,),
        in_specs=[
            pl.BlockSpec((RBLK, D), lambda i: (i, 0)),
            pl.BlockSpec((D, 2 * D), lambda i: (0, 0)),
            pl.BlockSpec((1, 2 * D), lambda i: (0, 0)),
            pl.BlockSpec((2 * D, L), lambda i: (0, 0)),
            pl.BlockSpec((D, L), lambda i: (0, 0)),
        ],
        out_specs=[
            pl.BlockSpec((RBLK, L), lambda i: (i, 0)),
            pl.BlockSpec((RBLK, L), lambda i: (i, 0)),
        ],
        out_shape=[
            jax.ShapeDtypeStruct((N, L), f32),
            jax.ShapeDtypeStruct((N, L), f32),
        ],
    )(x, pt, bp, w1t, wrt)


NBUF = 8
GLEAD = NBUF // 2


def _edge_agg_body(y_hbm, src_hbm, dst_hbm, zer_hbm, out_hbm,
                   src_v, dst_v, rows, shared, gsem, ssem):
    c = lax.axis_index("c")
    s = lax.axis_index("s")
    w = c * NS + s
    pltpu.sync_copy(zer_hbm.at[pl.ds(s * RPT, RPT)], shared.at[pl.ds(s * RPT, RPT)])

    @pl.when(s == NS - 1)
    def _():
        pltpu.sync_copy(zer_hbm.at[pl.ds(NS * RPT, REM)], shared.at[pl.ds(NS * RPT, REM)])
    pltpu.sync_copy(src_hbm.at[w], src_v)
    pltpu.sync_copy(dst_hbm.at[w], dst_v)
    plsc.subcore_barrier()

    for j in range(GLEAD):
        b = j % NBUF
        pltpu.async_copy(y_hbm.at[src_v.at[j]], rows.at[b], gsem.at[b])
    for j in range(CHUNKS):
        jg = j + GLEAD
        if jg < CHUNKS:
            bg = jg % NBUF
            if jg >= NBUF:
                pltpu.make_async_copy(
                    rows.at[bg], shared.at[dst_v.at[jg - NBUF]], ssem.at[bg]).wait()
            pltpu.async_copy(y_hbm.at[src_v.at[jg]], rows.at[bg], gsem.at[bg])
        b = j % NBUF
        pltpu.make_async_copy(y_hbm.at[src_v.at[j]], rows.at[b], gsem.at[b]).wait()
        pltpu.async_copy(rows.at[b], shared.at[dst_v.at[j]], ssem.at[b], add=True)
    for j in range(CHUNKS - NBUF, CHUNKS):
        b = j % NBUF
        pltpu.make_async_copy(
            rows.at[b], shared.at[dst_v.at[j]], ssem.at[b]).wait()
    plsc.subcore_barrier()
    pltpu.sync_copy(shared.at[pl.ds(s * RPT, RPT)], out_hbm.at[c, pl.ds(s * RPT, RPT)])

    @pl.when(s == NS - 1)
    def _():
        pltpu.sync_copy(shared.at[pl.ds(NS * RPT, REM)], out_hbm.at[c, pl.ds(NS * RPT, REM)])


def _edge_agg(y, src3, dst3, zer):
    k = pl.kernel(
        _edge_agg_body,
        out_type=jax.ShapeDtypeStruct((NC, N, L), f32),
        mesh=plsc.VectorSubcoreMesh(core_axis_name="c", subcore_axis_name="s"),
        scratch_types=[
            pltpu.VMEM((CHUNKS, CLEN), i32),
            pltpu.VMEM((CHUNKS, CLEN), i32),
            pltpu.VMEM((NBUF, CLEN, L), f32),
            pltpu.VMEM_SHARED((N, L), f32),
            pltpu.SemaphoreType.DMA((NBUF,)),
            pltpu.SemaphoreType.DMA((NBUF,)),
        ],
        compiler_params=pltpu.CompilerParams(use_tc_tiling_on_sc=False),
    )
    return k(y, src3, dst3, zer)


def _mid_body(parts_ref, xr_ref, w2t_ref, wr2t_ref, blg_ref, y2_ref, xr2_ref):
    a = parts_ref[0] + parts_ref[1]
    blcat = blg_ref[0:1, :]
    gcat = blg_ref[1:2, :]
    bcat = blg_ref[2:3, :]
    lane = lax.broadcasted_iota(i32, (RBLK, L), 1)
    m_idx = lane < 2
    m_pitch = (lane >= 2) & (lane < 14)
    cnt = jnp.sum(jnp.where(lane == 14, a, 0.0), axis=1, keepdims=True)
    inv = 1.0 / jnp.maximum(cnt, 1.0)
    o = jnp.where(lane < 14, a * inv + blcat + xr_ref[...], 0.0)
    ssq_i = jnp.sum(jnp.where(m_idx, o * o, 0.0), axis=1, keepdims=True)
    ssq_p = jnp.sum(jnp.where(m_pitch, o * o, 0.0), axis=1, keepdims=True)
    nrm = jnp.sqrt(jnp.where(m_idx, ssq_i, ssq_p))
    o = o / jnp.maximum(nrm, 1e-12)
    o = jnp.maximum(o, 0.0)
    mean = jnp.where(m_idx,
                     jnp.sum(jnp.where(m_idx, o, 0.0), 1, keepdims=True) * (1.0 / 2.0),
                     jnp.sum(jnp.where(m_pitch, o, 0.0), 1, keepdims=True) * (1.0 / 12.0))
    dlt = o - mean
    var = jnp.where(m_idx,
                    jnp.sum(jnp.where(m_idx, dlt * dlt, 0.0), 1, keepdims=True) * (1.0 / 2.0),
                    jnp.sum(jnp.where(m_pitch, dlt * dlt, 0.0), 1, keepdims=True) * (1.0 / 12.0))
    h1 = jnp.where(lane < 14, dlt / jnp.sqrt(var + 1e-5) * gcat + bcat, 0.0)
    y2_ref[...] = jnp.dot(h1, w2t_ref[...], preferred_element_type=f32)
    xr2_ref[...] = (jnp.dot(h1, wr2t_ref[...], preferred_element_type=f32)
                    + jnp.where(lane == 14, inv, 0.0))


def _mid(parts, xr, w2t, wr2t, blg):
    return pl.pallas_call(
        _mid_body,
        grid=(GRID,),
        in_specs=[
            pl.BlockSpec((NC, RBLK, L), lambda i: (0, i, 0)),
            pl.BlockSpec((RBLK, L), lambda i: (i, 0)),
            pl.BlockSpec((L, L), lambda i: (0, 0)),
            pl.BlockSpec((L, L), lambda i: (0, 0)),
            pl.BlockSpec((3, L), lambda i: (0, 0)),
        ],
        out_specs=[
            pl.BlockSpec((RBLK, L), lambda i: (i, 0)),
            pl.BlockSpec((RBLK, L), lambda i: (i, 0)),
        ],
        out_shape=[
            jax.ShapeDtypeStruct((N, L), f32),
            jax.ShapeDtypeStruct((N, L), f32),
        ],
    )(parts, xr, w2t, wr2t, blg)


def _final_body(parts_ref, xr2_ref, bl2_ref, emb_ref):
    a = parts_ref[0] + parts_ref[1]
    xr2 = xr2_ref[...]
    lane = lax.broadcasted_iota(i32, (RBLK, L), 1)
    inv = jnp.sum(jnp.where(lane == 14, xr2, 0.0), axis=1, keepdims=True)
    e = jnp.where(lane < 14, a * inv + bl2_ref[...] + xr2, 0.0)
    scores = jnp.sum(jnp.where(lane < 2, e, 0.0), axis=1, keepdims=True)
    emb_ref[...] = e + jnp.where(lane == 15, scores, 0.0)


def _final(parts, xr2, bl2):
    return pl.pallas_call(
        _final_body,
        grid=(GRID,),
        in_specs=[
            pl.BlockSpec((NC, RBLK, L), lambda i: (0, i, 0)),
            pl.BlockSpec((RBLK, L), lambda i: (i, 0)),
            pl.BlockSpec((1, L), lambda i: (0, 0)),
        ],
        out_specs=pl.BlockSpec((RBLK, L), lambda i: (i, 0)),
        out_shape=jax.ShapeDtypeStruct((N, L), f32),
    )(parts, xr2, bl2)


def _select_body(emb_hbm, sidx_hbm, cid_hbm, out_hbm,
                 sidx_v, cid_v, sc_v, row_v, out_v, sem):
    c = lax.axis_index("c")
    s = lax.axis_index("s")

    @pl.when((c == 0) & (s == 0))
    def _():
        pltpu.sync_copy(sidx_hbm, sidx_v)
        pltpu.sync_copy(cid_hbm, cid_v)
        best_v = jnp.full((L,), -3e38, f32)
        best_n = jnp.zeros((L,), i32)
        best_p = jnp.full((L,), 2**30, i32)
        lane = lax.iota(i32, L)
        for j in range(NCOMP_PAD // 128):
            pltpu.async_copy(emb_hbm.at[sidx_v.at[j]], sc_v.at[j], sem)
        for j in range(NCOMP_PAD // 128):
            pltpu.make_async_copy(emb_hbm.at[sidx_v.at[j]], sc_v.at[j], sem).wait()
            for k in range(128 // L):
                v = sc_v.at[j][pl.ds(k * L, L)]
                cid = cid_v.at[j][pl.ds(k * L, L)]
                pos = lane + (j * 128 + k * L)
                upd = (v > best_v) | ((v == best_v) & (pos < best_p))
                best_v = jnp.where(upd, v, best_v)
                best_n = jnp.where(upd, cid, best_n)
                best_p = jnp.where(upd, pos, best_p)
        bv, bn, bp = best_v[0], best_n[0], best_p[0]
        for l in range(1, L):
            v = best_v[l]
            take = (v > bv) | ((v == bv) & (best_p[l] < bp))
            bv = jnp.where(take, v, bv)
            bn = jnp.where(take, best_n[l], bn)
            bp = jnp.where(take, best_p[l], bp)
        pltpu.sync_copy(emb_hbm.at[pl.ds(bn * L, L)], row_v)
        rv = row_v[pl.ds(0, L)]
        pv = rv[2]
        pi = jnp.int32(0)
        for l in range(3, 14):
            v = rv[l]
            take = v > pv
            pv = jnp.where(take, v, pv)
            pi = jnp.where(take, jnp.int32(l - 2), pi)
        out_v[...] = jnp.where(lane == 0, bn, 0) + jnp.where(lane == 1, pi, 0)
        pltpu.sync_copy(out_v, out_hbm)


def _select(emb_flat, sidx, cid):
    k = pl.kernel(
        _select_body,
        out_type=jax.ShapeDtypeStruct((L,), i32),
        mesh=plsc.VectorSubcoreMesh(core_axis_name="c", subcore_axis_name="s"),
        scratch_types=[
            pltpu.VMEM((NCOMP_PAD // 128, 128), i32),
            pltpu.VMEM((NCOMP_PAD // 128, 128), i32),
            pltpu.VMEM((NCOMP_PAD // 128, 128), f32),
            pltpu.VMEM((L,), f32),
            pltpu.VMEM((L,), i32),
            pltpu.SemaphoreType.DMA,
        ],
    )
    return k(emb_flat, sidx, cid)


def kernel(x, edge_index, ts_beats, divs_pq, onset_div, duration_div,
           not_removed_notes, computation_notes, target,
           params_op, params_idx, params_pitch):
    del ts_beats, divs_pq, onset_div, duration_div, not_removed_notes
    del target, params_op
    pi, pp = params_idx, params_pitch

    pt = jnp.concatenate([pi['c1']['Wp'], pp['c1']['Wp']], axis=0).T
    bp = jnp.concatenate([pi['c1']['bp'], pp['c1']['bp']]).reshape(1, 2 * D)
    w1t = (jnp.zeros((2 * D, L), f32)
           .at[:D, 0:2].set(pi['c1']['Wl'].T)
           .at[D:, 2:14].set(pp['c1']['Wl'].T))
    wrt = (jnp.zeros((D, L), f32)
           .at[:, 0:2].set(pi['c1']['Wr'].T)
           .at[:, 2:14].set(pp['c1']['Wr'].T))
    blg = (jnp.zeros((3, L), f32)
           .at[0, 0:2].set(pi['c1']['bl']).at[0, 2:14].set(pp['c1']['bl'])
           .at[1, 0:2].set(pi['ln_g']).at[1, 2:14].set(pp['ln_g'])
           .at[2, 0:2].set(pi['ln_b']).at[2, 2:14].set(pp['ln_b']))
    w2t = (jnp.zeros((L, L), f32)
           .at[0:2, 0:2].set(pi['c2']['Wl'].T)
           .at[2:14, 2:14].set(pp['c2']['Wl'].T))
    wr2t = (jnp.zeros((L, L), f32)
            .at[0:2, 0:2].set(pi['c2']['Wr'].T)
            .at[2:14, 2:14].set(pp['c2']['Wr'].T))
    bl2 = (jnp.zeros((1, L), f32)
           .at[0, 0:2].set(pi['c2']['bl']).at[0, 2:14].set(pp['c2']['bl']))

    src3 = edge_index[0].astype(i32).reshape(NW, CHUNKS, CLEN)
    dst3 = edge_index[1].astype(i32).reshape(NW, CHUNKS, CLEN)
    zer = jnp.zeros((N, L), f32)

    comp = jnp.sort(computation_notes).astype(i32)
    comp_pad = jnp.concatenate([comp, jnp.broadcast_to(comp[0], (NCOMP_PAD - NCOMP,))])
    cid = comp_pad.reshape(NCOMP_PAD // 128, 128)
    sidx = cid * L + 15

    y1, xr = _proj(x, pt, bp, w1t, wrt)
    parts1 = _edge_agg(y1, src3, dst3, zer)
    y2, xr2 = _mid(parts1, xr, w2t, wr2t, blg)
    parts2 = _edge_agg(y2, src3, dst3, zer)
    emb = _final(parts2, xr2, bl2)
    sel = _select(emb.reshape(N * L), sidx, cid)

    note_index = sel[0]
    new_pitch = sel[1]
    return x.at[note_index, :12].set(jax.nn.one_hot(new_pitch, 12, dtype=x.dtype))


# speedup vs baseline: 38.1336x; 1.0508x over previous
import functools

import jax
import jax.numpy as jnp
from jax import lax
from jax.experimental import pallas as pl
from jax.experimental.pallas import tpu as pltpu
from jax.experimental.pallas import tpu_sc as plsc

N = 10000
D = 128
E = 320000
L = 16
NC = 2
NS = 16
NW = NC * NS
EPW = E // NW
CHUNKS = 80
CLEN = EPW // CHUNKS
RPT = 624
REM = N - NS * RPT
RBLK = 2000
GRID = N // RBLK
NCOMP = 1000
NCOMP_PAD = 1024

f32 = jnp.float32
i32 = jnp.int32


def _proj_body(x_ref, pt_ref, bp_ref, w1t_ref, wrt_ref, y1_ref, xr_ref):
    xb = x_ref[...]
    p = jnp.maximum(jnp.dot(xb, pt_ref[...], preferred_element_type=f32) + bp_ref[...], 0.0)
    y1 = jnp.dot(p, w1t_ref[...], preferred_element_type=f32)
    lane = lax.broadcasted_iota(i32, (RBLK, L), 1)
    y1_ref[...] = y1 + jnp.where(lane == 14, 1.0, 0.0)
    xr_ref[...] = jnp.dot(xb, wrt_ref[...], preferred_element_type=f32)


def _proj(x, pt, bp, w1t, wrt):
    return pl.pallas_call(
        _proj_body,
        grid=(GRID,),
        in_specs=[
            pl.BlockSpec((RBLK, D), lambda i: (i, 0)),
            pl.BlockSpec((D, 2 * D), lambda i: (0, 0)),
            pl.BlockSpec((1, 2 * D), lambda i: (0, 0)),
            pl.BlockSpec((2 * D, L), lambda i: (0, 0)),
            pl.BlockSpec((D, L), lambda i: (0, 0)),
        ],
        out_specs=[
            pl.BlockSpec((RBLK, L), lambda i: (i, 0)),
            pl.BlockSpec((RBLK, L), lambda i: (i, 0)),
        ],
        out_shape=[
            jax.ShapeDtypeStruct((N, L), f32),
            jax.ShapeDtypeStruct((N, L), f32),
        ],
    )(x, pt, bp, w1t, wrt)


NBUF = 8
GLEAD = NBUF // 2


def _edge_agg_body(y_hbm, src_hbm, dst_hbm, zer_hbm, out_hbm,
                   src_v, dst_v, rows, shared, gsem, ssem):
    c = lax.axis_index("c")
    s = lax.axis_index("s")
    w = c * NS + s
    pltpu.sync_copy(zer_hbm.at[pl.ds(s * RPT, RPT)], shared.at[pl.ds(s * RPT, RPT)])

    @pl.when(s == NS - 1)
    def _():
        pltpu.sync_copy(zer_hbm.at[pl.ds(NS * RPT, REM)], shared.at[pl.ds(NS * RPT, REM)])
    pltpu.sync_copy(src_hbm.at[w], src_v)
    pltpu.sync_copy(dst_hbm.at[w], dst_v)
    plsc.subcore_barrier()

    for j in range(GLEAD):
        b = j % NBUF
        pltpu.async_copy(y_hbm.at[src_v.at[j]], rows.at[b], gsem.at[b])
    for j in range(CHUNKS):
        jg = j + GLEAD
        if jg < CHUNKS:
            bg = jg % NBUF
            if jg >= NBUF:
                pltpu.make_async_copy(
                    rows.at[bg], shared.at[dst_v.at[jg - NBUF]], ssem.at[bg]).wait()
            pltpu.async_copy(y_hbm.at[src_v.at[jg]], rows.at[bg], gsem.at[bg])
        b = j % NBUF
        pltpu.make_async_copy(y_hbm.at[src_v.at[j]], rows.at[b], gsem.at[b]).wait()
        pltpu.async_copy(rows.at[b], shared.at[dst_v.at[j]], ssem.at[b], add=True)
    for j in range(CHUNKS - NBUF, CHUNKS):
        b = j % NBUF
        pltpu.make_async_copy(
            rows.at[b], shared.at[dst_v.at[j]], ssem.at[b]).wait()
    plsc.subcore_barrier()
    pltpu.sync_copy(shared.at[pl.ds(s * RPT, RPT)], out_hbm.at[c, pl.ds(s * RPT, RPT)])

    @pl.when(s == NS - 1)
    def _():
        pltpu.sync_copy(shared.at[pl.ds(NS * RPT, REM)], out_hbm.at[c, pl.ds(NS * RPT, REM)])


def _edge_agg(y, src3, dst3, zer):
    k = pl.kernel(
        _edge_agg_body,
        out_type=jax.ShapeDtypeStruct((NC, N, L), f32),
        mesh=plsc.VectorSubcoreMesh(core_axis_name="c", subcore_axis_name="s"),
        scratch_types=[
            pltpu.VMEM((CHUNKS, CLEN), i32),
            pltpu.VMEM((CHUNKS, CLEN), i32),
            pltpu.VMEM((NBUF, CLEN, L), f32),
            pltpu.VMEM_SHARED((N, L), f32),
            pltpu.SemaphoreType.DMA((NBUF,)),
            pltpu.SemaphoreType.DMA((NBUF,)),
        ],
        compiler_params=pltpu.CompilerParams(use_tc_tiling_on_sc=False),
    )
    return k(y, src3, dst3, zer)


def _mid_body(parts_ref, xr_ref, w2t_ref, wr2t_ref, blg_ref, y2_ref, xr2_ref):
    a = parts_ref[0] + parts_ref[1]
    blcat = blg_ref[0:1, :]
    gcat = blg_ref[1:2, :]
    bcat = blg_ref[2:3, :]
    lane = lax.broadcasted_iota(i32, (RBLK, L), 1)
    m_idx = lane < 2
    m_pitch = (lane >= 2) & (lane < 14)
    cnt = jnp.sum(jnp.where(lane == 14, a, 0.0), axis=1, keepdims=True)
    inv = 1.0 / jnp.maximum(cnt, 1.0)
    o = jnp.where(lane < 14, a * inv + blcat + xr_ref[...], 0.0)
    ssq_i = jnp.sum(jnp.where(m_idx, o * o, 0.0), axis=1, keepdims=True)
    ssq_p = jnp.sum(jnp.where(m_pitch, o * o, 0.0), axis=1, keepdims=True)
    nrm = jnp.sqrt(jnp.where(m_idx, ssq_i, ssq_p))
    o = o / jnp.maximum(nrm, 1e-12)
    o = jnp.maximum(o, 0.0)
    mean = jnp.where(m_idx,
                     jnp.sum(jnp.where(m_idx, o, 0.0), 1, keepdims=True) * (1.0 / 2.0),
                     jnp.sum(jnp.where(m_pitch, o, 0.0), 1, keepdims=True) * (1.0 / 12.0))
    dlt = o - mean
    var = jnp.where(m_idx,
                    jnp.sum(jnp.where(m_idx, dlt * dlt, 0.0), 1, keepdims=True) * (1.0 / 2.0),
                    jnp.sum(jnp.where(m_pitch, dlt * dlt, 0.0), 1, keepdims=True) * (1.0 / 12.0))
    h1 = jnp.where(lane < 14, dlt / jnp.sqrt(var + 1e-5) * gcat + bcat, 0.0)
    y2_ref[...] = jnp.dot(h1, w2t_ref[...], preferred_element_type=f32)
    xr2_ref[...] = (jnp.dot(h1, wr2t_ref[...], preferred_element_type=f32)
                    + jnp.where(lane == 14, inv, 0.0))


def _mid(parts, xr, w2t, wr2t, blg):
    return pl.pallas_call(
        _mid_body,
        grid=(GRID,),
        in_specs=[
            pl.BlockSpec((NC, RBLK, L), lambda i: (0, i, 0)),
            pl.BlockSpec((RBLK, L), lambda i: (i, 0)),
            pl.BlockSpec((L, L), lambda i: (0, 0)),
            pl.BlockSpec((L, L), lambda i: (0, 0)),
            pl.BlockSpec((3, L), lambda i: (0, 0)),
        ],
        out_specs=[
            pl.BlockSpec((RBLK, L), lambda i: (i, 0)),
            pl.BlockSpec((RBLK, L), lambda i: (i, 0)),
        ],
        out_shape=[
            jax.ShapeDtypeStruct((N, L), f32),
            jax.ShapeDtypeStruct((N, L), f32),
        ],
    )(parts, xr, w2t, wr2t, blg)


NCH = NCOMP_PAD // 128
NSTR = 7


def _select_body(p2_hbm, xr2_hbm, idx_hbm, cid_hbm, bl2_hbm, out_hbm,
                 idx_v, cid_v, g_v, bl2_v, row_v, out_v, sem, rsem):
    c = lax.axis_index("c")
    s = lax.axis_index("s")

    @pl.when((c == 0) & (s == 0))
    def _():
        pltpu.sync_copy(idx_hbm, idx_v)
        pltpu.sync_copy(cid_hbm, cid_v)
        pltpu.sync_copy(bl2_hbm, bl2_v)
        for v in range(4):
            for j in range(NCH):
                pltpu.async_copy(p2_hbm.at[idx_v.at[v, j]], g_v.at[v, j], sem)
        for v in range(4, NSTR):
            for j in range(NCH):
                pltpu.async_copy(xr2_hbm.at[idx_v.at[v, j]], g_v.at[v, j], sem)
        for v in range(NSTR):
            for j in range(NCH):
                pltpu.make_async_copy(p2_hbm.at[idx_v.at[v, j]], g_v.at[v, j], sem).wait()
        bl2 = bl2_v[pl.ds(0, L)]
        bsum = bl2[0] + bl2[1]
        best_v = jnp.full((L,), -3e38, f32)
        best_n = jnp.zeros((L,), i32)
        best_p = jnp.full((L,), 2**30, i32)
        lane = lax.iota(i32, L)
        for j in range(NCH):
            for k in range(128 // L):
                sl = pl.ds(k * L, L)
                p0a = g_v.at[0, j][sl]
                p0b = g_v.at[1, j][sl]
                p1a = g_v.at[2, j][sl]
                p1b = g_v.at[3, j][sl]
                xa = g_v.at[4, j][sl]
                xb = g_v.at[5, j][sl]
                xinv = g_v.at[6, j][sl]
                v = (p0a + p1a + p0b + p1b) * xinv + xa + xb + bsum
                cid = cid_v.at[j][sl]
                pos = lane + (j * 128 + k * L)
                upd = (v > best_v) | ((v == best_v) & (pos < best_p))
                best_v = jnp.where(upd, v, best_v)
                best_n = jnp.where(upd, cid, best_n)
                best_p = jnp.where(upd, pos, best_p)
        bv, bn, bp = best_v[0], best_n[0], best_p[0]
        for l in range(1, L):
            v = best_v[l]
            take = (v > bv) | ((v == bv) & (best_p[l] < bp))
            bv = jnp.where(take, v, bv)
            bn = jnp.where(take, best_n[l], bn)
            bp = jnp.where(take, best_p[l], bp)
        pltpu.async_copy(p2_hbm.at[pl.ds(bn * L, L)], row_v.at[0], rsem)
        pltpu.async_copy(p2_hbm.at[pl.ds(N * L + bn * L, L)], row_v.at[1], rsem)
        pltpu.async_copy(xr2_hbm.at[pl.ds(bn * L, L)], row_v.at[2], rsem)
        for r in range(3):
            pltpu.make_async_copy(p2_hbm.at[pl.ds(0, L)], row_v.at[r], rsem).wait()
        xrow = row_v.at[2][pl.ds(0, L)]
        inv_s = jnp.full((L,), xrow[14], f32)
        rv = ((row_v.at[0][pl.ds(0, L)] + row_v.at[1][pl.ds(0, L)]) * inv_s
              + bl2 + xrow)
        pv = rv[2]
        pi = jnp.int32(0)
        for l in range(3, 14):
            v = rv[l]
            take = v > pv
            pv = jnp.where(take, v, pv)
            pi = jnp.where(take, jnp.int32(l - 2), pi)
        out_v[...] = jnp.where(lane == 0, bn, 0) + jnp.where(lane == 1, pi, 0)
        pltpu.sync_copy(out_v, out_hbm)


def _select(p2_flat, xr2_flat, idx, cid, bl2):
    k = pl.kernel(
        _select_body,
        out_type=jax.ShapeDtypeStruct((L,), i32),
        mesh=plsc.VectorSubcoreMesh(core_axis_name="c", subcore_axis_name="s"),
        scratch_types=[
            pltpu.VMEM((NSTR, NCH, 128), i32),
            pltpu.VMEM((NCH, 128), i32),
            pltpu.VMEM((NSTR, NCH, 128), f32),
            pltpu.VMEM((L,), f32),
            pltpu.VMEM((3, L), f32),
            pltpu.VMEM((L,), i32),
            pltpu.SemaphoreType.DMA,
            pltpu.SemaphoreType.DMA,
        ],
    )
    return k(p2_flat, xr2_flat, idx, cid, bl2)


def kernel(x, edge_index, ts_beats, divs_pq, onset_div, duration_div,
           not_removed_notes, computation_notes, target,
           params_op, params_idx, params_pitch):
    del ts_beats, divs_pq, onset_div, duration_div, not_removed_notes
    del target, params_op
    pi, pp = params_idx, params_pitch

    pt = jnp.concatenate([pi['c1']['Wp'], pp['c1']['Wp']], axis=0).T
    bp = jnp.concatenate([pi['c1']['bp'], pp['c1']['bp']]).reshape(1, 2 * D)
    w1t = (jnp.zeros((2 * D, L), f32)
           .at[:D, 0:2].set(pi['c1']['Wl'].T)
           .at[D:, 2:14].set(pp['c1']['Wl'].T))
    wrt = (jnp.zeros((D, L), f32)
           .at[:, 0:2].set(pi['c1']['Wr'].T)
           .at[:, 2:14].set(pp['c1']['Wr'].T))
    blg = (jnp.zeros((3, L), f32)
           .at[0, 0:2].set(pi['c1']['bl']).at[0, 2:14].set(pp['c1']['bl'])
           .at[1, 0:2].set(pi['ln_g']).at[1, 2:14].set(pp['ln_g'])
           .at[2, 0:2].set(pi['ln_b']).at[2, 2:14].set(pp['ln_b']))
    w2t = (jnp.zeros((L, L), f32)
           .at[0:2, 0:2].set(pi['c2']['Wl'].T)
           .at[2:14, 2:14].set(pp['c2']['Wl'].T))
    wr2t = (jnp.zeros((L, L), f32)
            .at[0:2, 0:2].set(pi['c2']['Wr'].T)
            .at[2:14, 2:14].set(pp['c2']['Wr'].T))
    bl2 = (jnp.zeros((L,), f32)
           .at[0:2].set(pi['c2']['bl']).at[2:14].set(pp['c2']['bl']))

    src3 = edge_index[0].astype(i32).reshape(NW, CHUNKS, CLEN)
    dst3 = edge_index[1].astype(i32).reshape(NW, CHUNKS, CLEN)
    zer = jnp.zeros((N, L), f32)

    comp = computation_notes.astype(i32)
    comp_pad = jnp.concatenate([comp, jnp.broadcast_to(comp[0], (NCOMP_PAD - NCOMP,))])
    cid = comp_pad.reshape(NCH, 128)
    base = cid * L
    idx = jnp.stack([base, base + 1, N * L + base, N * L + base + 1,
                     base, base + 1, base + 14])

    y1, xr = _proj(x, pt, bp, w1t, wrt)
    parts1 = _edge_agg(y1, src3, dst3, zer)
    y2, xr2 = _mid(parts1, xr, w2t, wr2t, blg)
    parts2 = _edge_agg(y2, src3, dst3, zer)
    sel = _select(parts2.reshape(NC * N * L), xr2.reshape(N * L), idx, cid, bl2)

    note_index = sel[0]
    new_pitch = sel[1]
    return x.at[note_index, :12].set(jax.nn.one_hot(new_pitch, 12, dtype=x.dtype))


# speedup vs baseline: 46.9848x; 1.2321x over previous
import functools

import jax
import jax.numpy as jnp
from jax import lax
from jax.experimental import pallas as pl
from jax.experimental.pallas import tpu as pltpu
from jax.experimental.pallas import tpu_sc as plsc

import numpy as _np

N = 10000
D = 128
E = 320000
L = 16
NC = 2
NS = 16
NW = NC * NS
CLEN = 128
NCHK = E // CLEN
CHUNKS = NCHK // NW
XTRA = NCHK - NW * CHUNKS
RPT = 624
REM = N - NS * RPT
RBLK = 2000
PBLK = RBLK // 8
GRID = N // RBLK
NP = N // 8
NCOMP = 1000
NCOMP_PAD = 1024

f32 = jnp.float32
i32 = jnp.int32
bf16 = jnp.bfloat16

_b = _np.zeros((L, L), _np.float32)
_b[14, :] = 1.0
_SCNT = _np.kron(_np.eye(8, dtype=_np.float32), _b)
_b2 = _np.zeros((L, L), _np.float32)
_b2[0:2, 0:2] = 1.0
_b2[2:14, 2:14] = 1.0
_S2 = _np.kron(_np.eye(8, dtype=_np.float32), _b2)
_bm = _b2 / _np.concatenate([_np.full(2, 2.0), _np.full(12, 12.0), _np.ones(2)]).astype(_np.float32)
_SM = _np.kron(_np.eye(8, dtype=_np.float32), _bm)


def _proj_body(x_ref, pt_ref, bp_ref, w1t_ref, wrt_ref, y1_ref, xr_ref):
    xb = x_ref[...].astype(bf16)
    p = jnp.maximum(jnp.dot(xb, pt_ref[...], preferred_element_type=f32) + bp_ref[...], 0.0)
    y1 = jnp.dot(p.astype(bf16), w1t_ref[...], preferred_element_type=f32)
    lane = lax.broadcasted_iota(i32, (RBLK, L), 1)
    y1_ref[...] = y1 + jnp.where(lane == 14, 1.0, 0.0)
    xr_ref[...] = jnp.dot(xb, wrt_ref[...], preferred_element_type=f32)


def _proj(x, pt, bp, w1t, wrt):
    return pl.pallas_call(
        _proj_body,
        grid=(GRID,),
        in_specs=[
            pl.BlockSpec((RBLK, D), lambda i: (i, 0)),
            pl.BlockSpec((D, 2 * D), lambda i: (0, 0)),
            pl.BlockSpec((1, 2 * D), lambda i: (0, 0)),
            pl.BlockSpec((2 * D, L), lambda i: (0, 0)),
            pl.BlockSpec((D, L), lambda i: (0, 0)),
        ],
        out_specs=[
            pl.BlockSpec((RBLK, L), lambda i: (i, 0)),
            pl.BlockSpec((RBLK, L), lambda i: (i, 0)),
        ],
        out_shape=[
            jax.ShapeDtypeStruct((N, L), f32),
            jax.ShapeDtypeStruct((N, L), f32),
        ],
    )(x, pt, bp, w1t, wrt)


NBUF = 8
GLEAD = NBUF // 2


def _edge_agg_body(y_hbm, edge_hbm, zer_hbm, out_hbm,
                   src_v, dst_v, srcx_v, dstx_v, rows, shared, gsem, ssem):
    c = lax.axis_index("c")
    s = lax.axis_index("s")
    w = c * NS + s
    pltpu.sync_copy(zer_hbm.at[pl.ds(s * RPT, RPT)], shared.at[pl.ds(s * RPT, RPT)])

    @pl.when(s == NS - 1)
    def _():
        pltpu.sync_copy(zer_hbm.at[pl.ds(NS * RPT, REM)], shared.at[pl.ds(NS * RPT, REM)])
    pltpu.sync_copy(edge_hbm.at[0, pl.ds(w * CHUNKS, CHUNKS)], src_v)
    pltpu.sync_copy(edge_hbm.at[1, pl.ds(w * CHUNKS, CHUNKS)], dst_v)

    @pl.when(w < XTRA)
    def _():
        pltpu.sync_copy(edge_hbm.at[0, pl.ds(NW * CHUNKS + w, 1)], srcx_v)
        pltpu.sync_copy(edge_hbm.at[1, pl.ds(NW * CHUNKS + w, 1)], dstx_v)
    plsc.subcore_barrier()

    for j in range(GLEAD):
        b = j % NBUF
        pltpu.async_copy(y_hbm.at[src_v.at[j]], rows.at[b], gsem.at[b])
    for j in range(CHUNKS):
        jg = j + GLEAD
        if jg < CHUNKS:
            bg = jg % NBUF
            if jg >= NBUF:
                pltpu.make_async_copy(
                    rows.at[bg], shared.at[dst_v.at[jg - NBUF]], ssem.at[bg]).wait()
            pltpu.async_copy(y_hbm.at[src_v.at[jg]], rows.at[bg], gsem.at[bg])
        b = j % NBUF
        pltpu.make_async_copy(y_hbm.at[src_v.at[j]], rows.at[b], gsem.at[b]).wait()
        pltpu.async_copy(rows.at[b], shared.at[dst_v.at[j]], ssem.at[b], add=True)
    for j in range(CHUNKS - NBUF, CHUNKS):
        b = j % NBUF
        pltpu.make_async_copy(
            rows.at[b], shared.at[dst_v.at[j]], ssem.at[b]).wait()

    @pl.when(w < XTRA)
    def _():
        pltpu.async_copy(y_hbm.at[srcx_v.at[0]], rows.at[0], gsem.at[0])
        pltpu.make_async_copy(y_hbm.at[srcx_v.at[0]], rows.at[0], gsem.at[0]).wait()
        pltpu.sync_copy(rows.at[0], shared.at[dstx_v.at[0]], add=True)
    plsc.subcore_barrier()
    pltpu.sync_copy(shared.at[pl.ds(s * RPT, RPT)], out_hbm.at[c, pl.ds(s * RPT, RPT)])

    @pl.when(s == NS - 1)
    def _():
        pltpu.sync_copy(shared.at[pl.ds(NS * RPT, REM)], out_hbm.at[c, pl.ds(NS * RPT, REM)])


def _edge_agg(y, edges, zer):
    k = pl.kernel(
        _edge_agg_body,
        out_type=jax.ShapeDtypeStruct((NC, N, L), f32),
        mesh=plsc.VectorSubcoreMesh(core_axis_name="c", subcore_axis_name="s"),
        scratch_types=[
            pltpu.VMEM((CHUNKS, CLEN), i32),
            pltpu.VMEM((CHUNKS, CLEN), i32),
            pltpu.VMEM((1, CLEN), i32),
            pltpu.VMEM((1, CLEN), i32),
            pltpu.VMEM((NBUF, CLEN, L), f32),
            pltpu.VMEM_SHARED((N, L), f32),
            pltpu.SemaphoreType.DMA((NBUF,)),
            pltpu.SemaphoreType.DMA((NBUF,)),
        ],
        compiler_params=pltpu.CompilerParams(use_tc_tiling_on_sc=False),
    )
    return k(y, edges, zer)


def _mid_body(parts_ref, xr_ref, w2t_ref, wr2t_ref, blg_ref,
              scnt_ref, s2_ref, sm_ref, y2_ref, xr2_ref):
    a = parts_ref[0, 0] + parts_ref[1, 0]
    blcat = blg_ref[0:1, :]
    gcat = blg_ref[1:2, :]
    bcat = blg_ref[2:3, :]
    lane = lax.broadcasted_iota(i32, (PBLK, D), 1) % L
    cnt = jnp.dot(jnp.where(lane == 14, a, 0.0), scnt_ref[...],
                  preferred_element_type=f32)
    inv = 1.0 / jnp.maximum(cnt, 1.0)
    o = jnp.where(lane < 14, a * inv + blcat + xr_ref[0], 0.0)
    nrm2 = jnp.dot(o * o, s2_ref[...], preferred_element_type=f32)
    o = o / jnp.maximum(jnp.sqrt(nrm2), 1e-12)
    o = jnp.maximum(o, 0.0)
    mean = jnp.dot(o, sm_ref[...], preferred_element_type=f32)
    dlt = o - mean
    var = jnp.dot(dlt * dlt, sm_ref[...], preferred_element_type=f32)
    h1 = jnp.where(lane < 14, dlt * lax.rsqrt(var + 1e-5) * gcat + bcat, 0.0)
    y2_ref[0] = jnp.dot(h1, w2t_ref[...], preferred_element_type=f32)
    xr2_ref[0] = (jnp.dot(h1, wr2t_ref[...], preferred_element_type=f32)
                  + jnp.where(lane == 14, inv, 0.0))


def _mid(parts, xr, w2t, wr2t, blg, scnt, s2, sm):
    return pl.pallas_call(
        _mid_body,
        grid=(GRID,),
        in_specs=[
            pl.BlockSpec((NC, 1, PBLK, D), lambda i: (0, i, 0, 0)),
            pl.BlockSpec((1, PBLK, D), lambda i: (i, 0, 0)),
            pl.BlockSpec((D, D), lambda i: (0, 0)),
            pl.BlockSpec((D, D), lambda i: (0, 0)),
            pl.BlockSpec((3, D), lambda i: (0, 0)),
            pl.BlockSpec((D, D), lambda i: (0, 0)),
            pl.BlockSpec((D, D), lambda i: (0, 0)),
            pl.BlockSpec((D, D), lambda i: (0, 0)),
        ],
        out_specs=[
            pl.BlockSpec((1, PBLK, D), lambda i: (i, 0, 0)),
            pl.BlockSpec((1, PBLK, D), lambda i: (i, 0, 0)),
        ],
        out_shape=[
            jax.ShapeDtypeStruct((GRID, PBLK, D), f32),
            jax.ShapeDtypeStruct((GRID, PBLK, D), f32),
        ],
    )(parts, xr, w2t, wr2t, blg, scnt, s2, sm)


NCH = NCOMP_PAD // 128
NSTR = 7


def _select_body(p2_hbm, xr2_hbm, idx_hbm, cid_hbm, bl2_hbm, out_hbm,
                 idx_v, cid_v, g_v, bl2_v, row_v, out_v, sem, rsem):
    c = lax.axis_index("c")
    s = lax.axis_index("s")

    @pl.when((c == 0) & (s == 0))
    def _():
        pltpu.sync_copy(idx_hbm, idx_v)
        pltpu.sync_copy(cid_hbm, cid_v)
        pltpu.sync_copy(bl2_hbm, bl2_v)
        for v in range(4):
            for j in range(NCH):
                pltpu.async_copy(p2_hbm.at[idx_v.at[v, j]], g_v.at[v, j], sem)
        for v in range(4, NSTR):
            for j in range(NCH):
                pltpu.async_copy(xr2_hbm.at[idx_v.at[v, j]], g_v.at[v, j], sem)
        for v in range(NSTR):
            for j in range(NCH):
                pltpu.make_async_copy(p2_hbm.at[idx_v.at[v, j]], g_v.at[v, j], sem).wait()
        bl2 = bl2_v[pl.ds(0, L)]
        bsum = bl2[0] + bl2[1]
        best_v = jnp.full((L,), -3e38, f32)
        best_n = jnp.zeros((L,), i32)
        best_p = jnp.full((L,), 2**30, i32)
        lane = lax.iota(i32, L)
        for j in range(NCH):
            for k in range(128 // L):
                sl = pl.ds(k * L, L)
                p0a = g_v.at[0, j][sl]
                p0b = g_v.at[1, j][sl]
                p1a = g_v.at[2, j][sl]
                p1b = g_v.at[3, j][sl]
                xa = g_v.at[4, j][sl]
                xb = g_v.at[5, j][sl]
                xinv = g_v.at[6, j][sl]
                v = (p0a + p1a + p0b + p1b) * xinv + xa + xb + bsum
                cid = cid_v.at[j][sl]
                pos = lane + (j * 128 + k * L)
                upd = (v > best_v) | ((v == best_v) & (pos < best_p))
                best_v = jnp.where(upd, v, best_v)
                best_n = jnp.where(upd, cid, best_n)
                best_p = jnp.where(upd, pos, best_p)
        bv, bn, bp = best_v[0], best_n[0], best_p[0]
        for l in range(1, L):
            v = best_v[l]
            take = (v > bv) | ((v == bv) & (best_p[l] < bp))
            bv = jnp.where(take, v, bv)
            bn = jnp.where(take, best_n[l], bn)
            bp = jnp.where(take, best_p[l], bp)
        pltpu.async_copy(p2_hbm.at[pl.ds(bn * L, L)], row_v.at[0], rsem)
        pltpu.async_copy(p2_hbm.at[pl.ds(N * L + bn * L, L)], row_v.at[1], rsem)
        pltpu.async_copy(xr2_hbm.at[pl.ds(bn * L, L)], row_v.at[2], rsem)
        for r in range(3):
            pltpu.make_async_copy(p2_hbm.at[pl.ds(0, L)], row_v.at[r], rsem).wait()
        xrow = row_v.at[2][pl.ds(0, L)]
        inv_s = jnp.full((L,), xrow[14], f32)
        rv = ((row_v.at[0][pl.ds(0, L)] + row_v.at[1][pl.ds(0, L)]) * inv_s
              + bl2 + xrow)
        pv = rv[2]
        pi = jnp.int32(0)
        for l in range(3, 14):
            v = rv[l]
            take = v > pv
            pv = jnp.where(take, v, pv)
            pi = jnp.where(take, jnp.int32(l - 2), pi)
        out_v[...] = jnp.where(lane == 0, bn, 0) + jnp.where(lane == 1, pi, 0)
        pltpu.sync_copy(out_v, out_hbm)


def _select(p2_flat, xr2_flat, idx, cid, bl2):
    k = pl.kernel(
        _select_body,
        out_type=jax.ShapeDtypeStruct((L,), i32),
        mesh=plsc.VectorSubcoreMesh(core_axis_name="c", subcore_axis_name="s"),
        scratch_types=[
            pltpu.VMEM((NSTR, NCH, 128), i32),
            pltpu.VMEM((NCH, 128), i32),
            pltpu.VMEM((NSTR, NCH, 128), f32),
            pltpu.VMEM((L,), f32),
            pltpu.VMEM((3, L), f32),
            pltpu.VMEM((L,), i32),
            pltpu.SemaphoreType.DMA,
            pltpu.SemaphoreType.DMA,
        ],
    )
    return k(p2_flat, xr2_flat, idx, cid, bl2)


def kernel(x, edge_index, ts_beats, divs_pq, onset_div, duration_div,
           not_removed_notes, computation_notes, target,
           params_op, params_idx, params_pitch):
    del ts_beats, divs_pq, onset_div, duration_div, not_removed_notes
    del target, params_op
    pi, pp = params_idx, params_pitch

    pt = jnp.concatenate([pi['c1']['Wp'], pp['c1']['Wp']], axis=0).T
    bp = jnp.concatenate([pi['c1']['bp'], pp['c1']['bp']]).reshape(1, 2 * D)
    w1t = (jnp.zeros((2 * D, L), f32)
           .at[:D, 0:2].set(pi['c1']['Wl'].T)
           .at[D:, 2:14].set(pp['c1']['Wl'].T))
    wrt = (jnp.zeros((D, L), f32)
           .at[:, 0:2].set(pi['c1']['Wr'].T)
           .at[:, 2:14].set(pp['c1']['Wr'].T))
    blg = (jnp.zeros((3, L), f32)
           .at[0, 0:2].set(pi['c1']['bl']).at[0, 2:14].set(pp['c1']['bl'])
           .at[1, 0:2].set(pi['ln_g']).at[1, 2:14].set(pp['ln_g'])
           .at[2, 0:2].set(pi['ln_b']).at[2, 2:14].set(pp['ln_b']))
    blg = jnp.tile(blg, (1, 8))
    w2t = (jnp.zeros((L, L), f32)
           .at[0:2, 0:2].set(pi['c2']['Wl'].T)
           .at[2:14, 2:14].set(pp['c2']['Wl'].T))
    wr2t = (jnp.zeros((L, L), f32)
            .at[0:2, 0:2].set(pi['c2']['Wr'].T)
            .at[2:14, 2:14].set(pp['c2']['Wr'].T))
    bl2 = (jnp.zeros((L,), f32)
           .at[0:2].set(pi['c2']['bl']).at[2:14].set(pp['c2']['bl']))

    edges = edge_index.astype(i32).reshape(2, NCHK, CLEN)
    zer = jnp.zeros((N, L), f32)

    comp = computation_notes.astype(i32)
    comp_pad = jnp.concatenate([comp, jnp.broadcast_to(comp[0], (NCOMP_PAD - NCOMP,))])
    cid = comp_pad.reshape(NCH, 128)
    base = cid * L
    idx = jnp.stack([base, base + 1, N * L + base, N * L + base + 1,
                     base, base + 1, base + 14])

    scnt = jnp.asarray(_SCNT)
    s2 = jnp.asarray(_S2)
    sm = jnp.asarray(_SM)

    y1, xr = _proj(x, pt.astype(bf16), bp, w1t.astype(bf16), wrt.astype(bf16))
    parts1 = _edge_agg(y1.reshape(N, L), edges, zer)
    eye8 = jnp.asarray(_np.eye(8, dtype=_np.float32))
    w2b = jnp.kron(eye8, w2t)
    wr2b = jnp.kron(eye8, wr2t)
    y2, xr2 = _mid(parts1.reshape(NC, GRID, PBLK, D), xr.reshape(GRID, PBLK, D),
                   w2b, wr2b, blg, scnt, s2, sm)
    parts2 = _edge_agg(y2.reshape(N, L), edges, zer)
    sel = _select(parts2.reshape(NC * N * L), xr2.reshape(N * L), idx, cid, bl2)

    note_index = sel[0]
    new_pitch = sel[1]
    return x.at[note_index, :12].set(jax.nn.one_hot(new_pitch, 12, dtype=x.dtype))


# speedup vs baseline: 48.7701x; 1.0380x over previous
import functools

import jax
import jax.numpy as jnp
from jax import lax
from jax.experimental import pallas as pl
from jax.experimental.pallas import tpu as pltpu
from jax.experimental.pallas import tpu_sc as plsc

import numpy as _np

N = 10000
D = 128
E = 320000
L = 16
NC = 2
NS = 16
NW = NC * NS
CLEN = 128
NCHK = E // CLEN
CHUNKS = NCHK // NW
XTRA = NCHK - NW * CHUNKS
RPT = 624
REM = N - NS * RPT
RBLK = 2000
PBLK = RBLK // 8
GRID = N // RBLK
NP = N // 8
NCOMP = 1000
NCOMP_PAD = 1024

f32 = jnp.float32
i32 = jnp.int32
bf16 = jnp.bfloat16

_b = _np.zeros((L, L), _np.float32)
_b[14, :] = 1.0
_SCNT = _np.kron(_np.eye(8, dtype=_np.float32), _b)
_b2 = _np.zeros((L, L), _np.float32)
_b2[0:2, 0:2] = 1.0
_b2[2:14, 2:14] = 1.0
_S2 = _np.kron(_np.eye(8, dtype=_np.float32), _b2)
_bm = _b2 / _np.concatenate([_np.full(2, 2.0), _np.full(12, 12.0), _np.ones(2)]).astype(_np.float32)
_SM = _np.kron(_np.eye(8, dtype=_np.float32), _bm)


def _proj_body(x_ref, pt_ref, bp_ref, w1t_ref, wrt_ref, y1_ref, xr_ref):
    xb = x_ref[...].astype(bf16)
    p = jnp.maximum(jnp.dot(xb, pt_ref[...], preferred_element_type=f32) + bp_ref[...], 0.0)
    y1 = jnp.dot(p.astype(bf16), w1t_ref[...], preferred_element_type=f32)
    lane = lax.broadcasted_iota(i32, (RBLK, L), 1)
    y1_ref[...] = y1 + jnp.where(lane == 14, 1.0, 0.0)
    xr_ref[...] = jnp.dot(xb, wrt_ref[...], preferred_element_type=f32)


def _proj(x, pt, bp, w1t, wrt):
    return pl.pallas_call(
        _proj_body,
        grid=(GRID,),
        in_specs=[
            pl.BlockSpec((RBLK, D), lambda i: (i, 0)),
            pl.BlockSpec((D, 2 * D), lambda i: (0, 0)),
            pl.BlockSpec((1, 2 * D), lambda i: (0, 0)),
            pl.BlockSpec((2 * D, L), lambda i: (0, 0)),
            pl.BlockSpec((D, L), lambda i: (0, 0)),
        ],
        out_specs=[
            pl.BlockSpec((RBLK, L), lambda i: (i, 0)),
            pl.BlockSpec((RBLK, L), lambda i: (i, 0)),
        ],
        out_shape=[
            jax.ShapeDtypeStruct((N, L), f32),
            jax.ShapeDtypeStruct((N, L), f32),
        ],
    )(x, pt, bp, w1t, wrt)


NBUF = 12
GLEAD = NBUF // 2


def _edge_agg_body(y_hbm, edge_hbm, zer_hbm, out_hbm,
                   src_v, dst_v, srcx_v, dstx_v, rows, shared, gsem, ssem):
    c = lax.axis_index("c")
    s = lax.axis_index("s")
    w = c * NS + s
    pltpu.sync_copy(zer_hbm.at[pl.ds(s * RPT, RPT)], shared.at[pl.ds(s * RPT, RPT)])

    @pl.when(s == NS - 1)
    def _():
        pltpu.sync_copy(zer_hbm.at[pl.ds(NS * RPT, REM)], shared.at[pl.ds(NS * RPT, REM)])
    pltpu.sync_copy(edge_hbm.at[0, pl.ds(w * CHUNKS, CHUNKS)], src_v)
    pltpu.sync_copy(edge_hbm.at[1, pl.ds(w * CHUNKS, CHUNKS)], dst_v)

    @pl.when(w < XTRA)
    def _():
        pltpu.sync_copy(edge_hbm.at[0, pl.ds(NW * CHUNKS + w, 1)], srcx_v)
        pltpu.sync_copy(edge_hbm.at[1, pl.ds(NW * CHUNKS + w, 1)], dstx_v)
    plsc.subcore_barrier()

    for j in range(GLEAD):
        b = j % NBUF
        pltpu.async_copy(y_hbm.at[src_v.at[j]], rows.at[b], gsem.at[b])
    for j in range(CHUNKS):
        jg = j + GLEAD
        if jg < CHUNKS:
            bg = jg % NBUF
            if jg >= NBUF:
                pltpu.make_async_copy(
                    rows.at[bg], shared.at[dst_v.at[jg - NBUF]], ssem.at[bg]).wait()
            pltpu.async_copy(y_hbm.at[src_v.at[jg]], rows.at[bg], gsem.at[bg])
        b = j % NBUF
        pltpu.make_async_copy(y_hbm.at[src_v.at[j]], rows.at[b], gsem.at[b]).wait()
        pltpu.async_copy(rows.at[b], shared.at[dst_v.at[j]], ssem.at[b], add=True)
    for j in range(CHUNKS - NBUF, CHUNKS):
        b = j % NBUF
        pltpu.make_async_copy(
            rows.at[b], shared.at[dst_v.at[j]], ssem.at[b]).wait()

    @pl.when(w < XTRA)
    def _():
        pltpu.async_copy(y_hbm.at[srcx_v.at[0]], rows.at[0], gsem.at[0])
        pltpu.make_async_copy(y_hbm.at[srcx_v.at[0]], rows.at[0], gsem.at[0]).wait()
        pltpu.sync_copy(rows.at[0], shared.at[dstx_v.at[0]], add=True)
    plsc.subcore_barrier()
    pltpu.sync_copy(shared.at[pl.ds(s * RPT, RPT)], out_hbm.at[c, pl.ds(s * RPT, RPT)])

    @pl.when(s == NS - 1)
    def _():
        pltpu.sync_copy(shared.at[pl.ds(NS * RPT, REM)], out_hbm.at[c, pl.ds(NS * RPT, REM)])


def _edge_agg(y, edges, zer):
    k = pl.kernel(
        _edge_agg_body,
        out_type=jax.ShapeDtypeStruct((NC, N, L), f32),
        mesh=plsc.VectorSubcoreMesh(core_axis_name="c", subcore_axis_name="s"),
        scratch_types=[
            pltpu.VMEM((CHUNKS, CLEN), i32),
            pltpu.VMEM((CHUNKS, CLEN), i32),
            pltpu.VMEM((1, CLEN), i32),
            pltpu.VMEM((1, CLEN), i32),
            pltpu.VMEM((NBUF, CLEN, L), f32),
            pltpu.VMEM_SHARED((N, L), f32),
            pltpu.SemaphoreType.DMA((NBUF,)),
            pltpu.SemaphoreType.DMA((NBUF,)),
        ],
        compiler_params=pltpu.CompilerParams(use_tc_tiling_on_sc=False),
    )
    return k(y, edges, zer)


def _mid_body(parts_ref, xr_ref, w2t_ref, wr2t_ref, blg_ref,
              scnt_ref, s2_ref, sm_ref, y2_ref, xr2_ref):
    a = parts_ref[0, 0] + parts_ref[1, 0]
    blcat = blg_ref[0:1, :]
    gcat = blg_ref[1:2, :]
    bcat = blg_ref[2:3, :]
    lane = lax.broadcasted_iota(i32, (PBLK, D), 1) % L
    cnt = jnp.dot(jnp.where(lane == 14, a, 0.0), scnt_ref[...],
                  preferred_element_type=f32)
    inv = 1.0 / jnp.maximum(cnt, 1.0)
    o = jnp.where(lane < 14, a * inv + blcat + xr_ref[0], 0.0)
    nrm2 = jnp.dot(o * o, s2_ref[...], preferred_element_type=f32)
    o = o / jnp.maximum(jnp.sqrt(nrm2), 1e-12)
    o = jnp.maximum(o, 0.0)
    mean = jnp.dot(o, sm_ref[...], preferred_element_type=f32)
    dlt = o - mean
    var = jnp.dot(dlt * dlt, sm_ref[...], preferred_element_type=f32)
    h1 = jnp.where(lane < 14, dlt * lax.rsqrt(var + 1e-5) * gcat + bcat, 0.0)
    y2_ref[0] = jnp.dot(h1, w2t_ref[...], preferred_element_type=f32)
    xr2_ref[0] = (jnp.dot(h1, wr2t_ref[...], preferred_element_type=f32)
                  + jnp.where(lane == 14, inv, 0.0))


def _mid(parts, xr, w2t, wr2t, blg, scnt, s2, sm):
    return pl.pallas_call(
        _mid_body,
        grid=(GRID,),
        in_specs=[
            pl.BlockSpec((NC, 1, PBLK, D), lambda i: (0, i, 0, 0)),
            pl.BlockSpec((1, PBLK, D), lambda i: (i, 0, 0)),
            pl.BlockSpec((D, D), lambda i: (0, 0)),
            pl.BlockSpec((D, D), lambda i: (0, 0)),
            pl.BlockSpec((3, D), lambda i: (0, 0)),
            pl.BlockSpec((D, D), lambda i: (0, 0)),
            pl.BlockSpec((D, D), lambda i: (0, 0)),
            pl.BlockSpec((D, D), lambda i: (0, 0)),
        ],
        out_specs=[
            pl.BlockSpec((1, PBLK, D), lambda i: (i, 0, 0)),
            pl.BlockSpec((1, PBLK, D), lambda i: (i, 0, 0)),
        ],
        out_shape=[
            jax.ShapeDtypeStruct((GRID, PBLK, D), f32),
            jax.ShapeDtypeStruct((GRID, PBLK, D), f32),
        ],
    )(parts, xr, w2t, wr2t, blg, scnt, s2, sm)


NCH = NCOMP_PAD // 128
NSTR = 7


def _select_body(p2_hbm, xr2_hbm, idx_hbm, cid_hbm, bl2_hbm, out_hbm,
                 idx_v, cid_v, g_v, bl2_v, row_v, out_v, sem, rsem):
    c = lax.axis_index("c")
    s = lax.axis_index("s")

    @pl.when((c == 0) & (s == 0))
    def _():
        pltpu.sync_copy(idx_hbm, idx_v)
        pltpu.sync_copy(cid_hbm, cid_v)
        pltpu.sync_copy(bl2_hbm, bl2_v)
        for v in range(4):
            for j in range(NCH):
                pltpu.async_copy(p2_hbm.at[idx_v.at[v, j]], g_v.at[v, j], sem)
        for v in range(4, NSTR):
            for j in range(NCH):
                pltpu.async_copy(xr2_hbm.at[idx_v.at[v, j]], g_v.at[v, j], sem)
        for v in range(NSTR):
            for j in range(NCH):
                pltpu.make_async_copy(p2_hbm.at[idx_v.at[v, j]], g_v.at[v, j], sem).wait()
        bl2 = bl2_v[pl.ds(0, L)]
        bsum = bl2[0] + bl2[1]
        best_v = jnp.full((L,), -3e38, f32)
        best_n = jnp.zeros((L,), i32)
        best_p = jnp.full((L,), 2**30, i32)
        lane = lax.iota(i32, L)
        for j in range(NCH):
            for k in range(128 // L):
                sl = pl.ds(k * L, L)
                p0a = g_v.at[0, j][sl]
                p0b = g_v.at[1, j][sl]
                p1a = g_v.at[2, j][sl]
                p1b = g_v.at[3, j][sl]
                xa = g_v.at[4, j][sl]
                xb = g_v.at[5, j][sl]
                xinv = g_v.at[6, j][sl]
                v = (p0a + p1a + p0b + p1b) * xinv + xa + xb + bsum
                cid = cid_v.at[j][sl]
                pos = lane + (j * 128 + k * L)
                upd = (v > best_v) | ((v == best_v) & (pos < best_p))
                best_v = jnp.where(upd, v, best_v)
                best_n = jnp.where(upd, cid, best_n)
                best_p = jnp.where(upd, pos, best_p)
        bv, bn, bp = best_v[0], best_n[0], best_p[0]
        for l in range(1, L):
            v = best_v[l]
            take = (v > bv) | ((v == bv) & (best_p[l] < bp))
            bv = jnp.where(take, v, bv)
            bn = jnp.where(take, best_n[l], bn)
            bp = jnp.where(take, best_p[l], bp)
        pltpu.async_copy(p2_hbm.at[pl.ds(bn * L, L)], row_v.at[0], rsem)
        pltpu.async_copy(p2_hbm.at[pl.ds(N * L + bn * L, L)], row_v.at[1], rsem)
        pltpu.async_copy(xr2_hbm.at[pl.ds(bn * L, L)], row_v.at[2], rsem)
        for r in range(3):
            pltpu.make_async_copy(p2_hbm.at[pl.ds(0, L)], row_v.at[r], rsem).wait()
        xrow = row_v.at[2][pl.ds(0, L)]
        inv_s = jnp.full((L,), xrow[14], f32)
        rv = ((row_v.at[0][pl.ds(0, L)] + row_v.at[1][pl.ds(0, L)]) * inv_s
              + bl2 + xrow)
        pv = rv[2]
        pi = jnp.int32(0)
        for l in range(3, 14):
            v = rv[l]
            take = v > pv
            pv = jnp.where(take, v, pv)
            pi = jnp.where(take, jnp.int32(l - 2), pi)
        out_v[...] = jnp.where(lane == 0, bn, 0) + jnp.where(lane == 1, pi, 0)
        pltpu.sync_copy(out_v, out_hbm)


def _select(p2_flat, xr2_flat, idx, cid, bl2):
    k = pl.kernel(
        _select_body,
        out_type=jax.ShapeDtypeStruct((L,), i32),
        mesh=plsc.VectorSubcoreMesh(core_axis_name="c", subcore_axis_name="s"),
        scratch_types=[
            pltpu.VMEM((NSTR, NCH, 128), i32),
            pltpu.VMEM((NCH, 128), i32),
            pltpu.VMEM((NSTR, NCH, 128), f32),
            pltpu.VMEM((L,), f32),
            pltpu.VMEM((3, L), f32),
            pltpu.VMEM((L,), i32),
            pltpu.SemaphoreType.DMA,
            pltpu.SemaphoreType.DMA,
        ],
    )
    return k(p2_flat, xr2_flat, idx, cid, bl2)


def kernel(x, edge_index, ts_beats, divs_pq, onset_div, duration_div,
           not_removed_notes, computation_notes, target,
           params_op, params_idx, params_pitch):
    del ts_beats, divs_pq, onset_div, duration_div, not_removed_notes
    del target, params_op
    pi, pp = params_idx, params_pitch

    pt = jnp.concatenate([pi['c1']['Wp'], pp['c1']['Wp']], axis=0).T
    bp = jnp.concatenate([pi['c1']['bp'], pp['c1']['bp']]).reshape(1, 2 * D)
    w1t = (jnp.zeros((2 * D, L), f32)
           .at[:D, 0:2].set(pi['c1']['Wl'].T)
           .at[D:, 2:14].set(pp['c1']['Wl'].T))
    wrt = (jnp.zeros((D, L), f32)
           .at[:, 0:2].set(pi['c1']['Wr'].T)
           .at[:, 2:14].set(pp['c1']['Wr'].T))
    blg = (jnp.zeros((3, L), f32)
           .at[0, 0:2].set(pi['c1']['bl']).at[0, 2:14].set(pp['c1']['bl'])
           .at[1, 0:2].set(pi['ln_g']).at[1, 2:14].set(pp['ln_g'])
           .at[2, 0:2].set(pi['ln_b']).at[2, 2:14].set(pp['ln_b']))
    blg = jnp.tile(blg, (1, 8))
    w2t = (jnp.zeros((L, L), f32)
           .at[0:2, 0:2].set(pi['c2']['Wl'].T)
           .at[2:14, 2:14].set(pp['c2']['Wl'].T))
    wr2t = (jnp.zeros((L, L), f32)
            .at[0:2, 0:2].set(pi['c2']['Wr'].T)
            .at[2:14, 2:14].set(pp['c2']['Wr'].T))
    bl2 = (jnp.zeros((L,), f32)
           .at[0:2].set(pi['c2']['bl']).at[2:14].set(pp['c2']['bl']))

    edges = edge_index.astype(i32).reshape(2, NCHK, CLEN)
    zer = jnp.zeros((N, L), f32)

    comp = computation_notes.astype(i32)
    comp_pad = jnp.concatenate([comp, jnp.broadcast_to(comp[0], (NCOMP_PAD - NCOMP,))])
    cid = comp_pad.reshape(NCH, 128)
    base = cid * L
    idx = jnp.stack([base, base + 1, N * L + base, N * L + base + 1,
                     base, base + 1, base + 14])

    scnt = jnp.asarray(_SCNT)
    s2 = jnp.asarray(_S2)
    sm = jnp.asarray(_SM)

    y1, xr = _proj(x, pt.astype(bf16), bp, w1t.astype(bf16), wrt.astype(bf16))
    parts1 = _edge_agg(y1.reshape(N, L), edges, zer)
    eye8 = jnp.asarray(_np.eye(8, dtype=_np.float32))
    w2b = jnp.kron(eye8, w2t)
    wr2b = jnp.kron(eye8, wr2t)
    y2, xr2 = _mid(parts1.reshape(NC, GRID, PBLK, D), xr.reshape(GRID, PBLK, D),
                   w2b, wr2b, blg, scnt, s2, sm)
    parts2 = _edge_agg(y2.reshape(N, L), edges, zer)
    sel = _select(parts2.reshape(NC * N * L), xr2.reshape(N * L), idx, cid, bl2)

    note_index = sel[0]
    new_pitch = sel[1]
    return x.at[note_index, :12].set(jax.nn.one_hot(new_pitch, 12, dtype=x.dtype))


# speedup vs baseline: 51.5919x; 1.0579x over previous
import functools

import jax
import jax.numpy as jnp
from jax import lax
from jax.experimental import pallas as pl
from jax.experimental.pallas import tpu as pltpu
from jax.experimental.pallas import tpu_sc as plsc

import numpy as _np

N = 10000
D = 128
E = 320000
L = 16
NC = 2
NS = 16
NW = NC * NS
CLEN = 128
NCHK = E // CLEN
CHUNKS = NCHK // NW
XTRA = NCHK - NW * CHUNKS
RPT = 624
REM = N - NS * RPT
RBLK = 2000
PBLK = RBLK // 8
GRID = N // RBLK
NP = N // 8
NCOMP = 1000
NCOMP_PAD = 1024

f32 = jnp.float32
i32 = jnp.int32
bf16 = jnp.bfloat16

_b = _np.zeros((L, L), _np.float32)
_b[14, :] = 1.0
_SCNT = _np.kron(_np.eye(8, dtype=_np.float32), _b)
_b2 = _np.zeros((L, L), _np.float32)
_b2[0:2, 0:2] = 1.0
_b2[2:14, 2:14] = 1.0
_S2 = _np.kron(_np.eye(8, dtype=_np.float32), _b2)
_bm = _b2 / _np.concatenate([_np.full(2, 2.0), _np.full(12, 12.0), _np.ones(2)]).astype(_np.float32)
_SM = _np.kron(_np.eye(8, dtype=_np.float32), _bm)


def _proj_body(x_ref, pt_ref, bp_ref, w1t_ref, wrt_ref, y1_ref, xr_ref):
    xb = x_ref[...].astype(bf16)
    p = jnp.maximum(jnp.dot(xb, pt_ref[...], preferred_element_type=f32) + bp_ref[...], 0.0)
    y1 = jnp.dot(p.astype(bf16), w1t_ref[...], preferred_element_type=f32)
    lane = lax.broadcasted_iota(i32, (RBLK, L), 1)
    y1_ref[...] = y1 + jnp.where(lane == 14, 1.0, 0.0)
    xr_ref[...] = jnp.dot(xb, wrt_ref[...], preferred_element_type=f32)


def _proj(x, pt, bp, w1t, wrt):
    return pl.pallas_call(
        _proj_body,
        grid=(GRID,),
        in_specs=[
            pl.BlockSpec((RBLK, D), lambda i: (i, 0)),
            pl.BlockSpec((D, 2 * D), lambda i: (0, 0)),
            pl.BlockSpec((1, 2 * D), lambda i: (0, 0)),
            pl.BlockSpec((2 * D, L), lambda i: (0, 0)),
            pl.BlockSpec((D, L), lambda i: (0, 0)),
        ],
        out_specs=[
            pl.BlockSpec((RBLK, L), lambda i: (i, 0)),
            pl.BlockSpec((RBLK, L), lambda i: (i, 0)),
        ],
        out_shape=[
            jax.ShapeDtypeStruct((N, L), f32),
            jax.ShapeDtypeStruct((N, L), f32),
        ],
    )(x, pt, bp, w1t, wrt)


NBUF = 12
GLEAD = NBUF // 2


def _edge_agg_body(y_hbm, edge_hbm, zer_hbm, out_hbm,
                   src_v, dst_v, srcx_v, dstx_v, rows, shared, gsem, ssem):
    c = lax.axis_index("c")
    s = lax.axis_index("s")
    w = c * NS + s
    pltpu.sync_copy(zer_hbm.at[pl.ds(s * RPT, RPT)], shared.at[pl.ds(s * RPT, RPT)])

    @pl.when(s == NS - 1)
    def _():
        pltpu.sync_copy(zer_hbm.at[pl.ds(NS * RPT, REM)], shared.at[pl.ds(NS * RPT, REM)])
    pltpu.sync_copy(edge_hbm.at[0, pl.ds(w * CHUNKS, CHUNKS)], src_v)
    pltpu.sync_copy(edge_hbm.at[1, pl.ds(w * CHUNKS, CHUNKS)], dst_v)

    @pl.when(w < XTRA)
    def _():
        pltpu.sync_copy(edge_hbm.at[0, pl.ds(NW * CHUNKS + w, 1)], srcx_v)
        pltpu.sync_copy(edge_hbm.at[1, pl.ds(NW * CHUNKS + w, 1)], dstx_v)
    plsc.subcore_barrier()

    for j in range(GLEAD):
        b = j % NBUF
        pltpu.async_copy(y_hbm.at[src_v.at[j]], rows.at[b], gsem.at[b])
    for j in range(CHUNKS):
        jg = j + GLEAD
        if jg < CHUNKS:
            bg = jg % NBUF
            if jg >= NBUF:
                pltpu.make_async_copy(
                    rows.at[bg], shared.at[dst_v.at[jg - NBUF]], ssem.at[bg]).wait()
            pltpu.async_copy(y_hbm.at[src_v.at[jg]], rows.at[bg], gsem.at[bg])
        b = j % NBUF
        pltpu.make_async_copy(y_hbm.at[src_v.at[j]], rows.at[b], gsem.at[b]).wait()
        pltpu.async_copy(rows.at[b], shared.at[dst_v.at[j]], ssem.at[b], add=True)
    for j in range(CHUNKS - NBUF, CHUNKS):
        b = j % NBUF
        pltpu.make_async_copy(
            rows.at[b], shared.at[dst_v.at[j]], ssem.at[b]).wait()

    @pl.when(w < XTRA)
    def _():
        pltpu.async_copy(y_hbm.at[srcx_v.at[0]], rows.at[0], gsem.at[0])
        pltpu.make_async_copy(y_hbm.at[srcx_v.at[0]], rows.at[0], gsem.at[0]).wait()
        pltpu.sync_copy(rows.at[0], shared.at[dstx_v.at[0]], add=True)
    plsc.subcore_barrier()
    pltpu.sync_copy(shared.at[pl.ds(s * RPT, RPT)], out_hbm.at[c, pl.ds(s * RPT, RPT)])

    @pl.when(s == NS - 1)
    def _():
        pltpu.sync_copy(shared.at[pl.ds(NS * RPT, REM)], out_hbm.at[c, pl.ds(NS * RPT, REM)])


def _edge_agg(y, edges, zer):
    k = pl.kernel(
        _edge_agg_body,
        out_type=jax.ShapeDtypeStruct((NC, N, L), f32),
        mesh=plsc.VectorSubcoreMesh(core_axis_name="c", subcore_axis_name="s"),
        scratch_types=[
            pltpu.VMEM((CHUNKS, CLEN), i32),
            pltpu.VMEM((CHUNKS, CLEN), i32),
            pltpu.VMEM((1, CLEN), i32),
            pltpu.VMEM((1, CLEN), i32),
            pltpu.VMEM((NBUF, CLEN, L), f32),
            pltpu.VMEM_SHARED((N, L), f32),
            pltpu.SemaphoreType.DMA((NBUF,)),
            pltpu.SemaphoreType.DMA((NBUF,)),
        ],
        compiler_params=pltpu.CompilerParams(use_tc_tiling_on_sc=False),
    )
    return k(y, edges, zer)


def _mid_body(parts_ref, xr_ref, w2t_ref, wr2t_ref, blg_ref,
              scnt_ref, s2_ref, sm_ref, y2_ref, xr2_ref):
    a = parts_ref[0, 0] + parts_ref[1, 0]
    blcat = blg_ref[0:1, :]
    gcat = blg_ref[1:2, :]
    bcat = blg_ref[2:3, :]
    lane = lax.broadcasted_iota(i32, (PBLK, D), 1) % L
    cnt = jnp.dot(jnp.where(lane == 14, a, 0.0), scnt_ref[...],
                  preferred_element_type=f32)
    inv = 1.0 / jnp.maximum(cnt, 1.0)
    o = jnp.where(lane < 14, a * inv + blcat + xr_ref[0], 0.0)
    nrm2 = jnp.dot(o * o, s2_ref[...], preferred_element_type=f32)
    o = o / jnp.maximum(jnp.sqrt(nrm2), 1e-12)
    o = jnp.maximum(o, 0.0)
    mean = jnp.dot(o, sm_ref[...], preferred_element_type=f32)
    dlt = o - mean
    var = jnp.dot(dlt * dlt, sm_ref[...], preferred_element_type=f32)
    h1 = jnp.where(lane < 14, dlt * lax.rsqrt(var + 1e-5) * gcat + bcat, 0.0)
    y2_ref[0] = jnp.dot(h1, w2t_ref[...], preferred_element_type=f32)
    xr2_ref[0] = (jnp.dot(h1, wr2t_ref[...], preferred_element_type=f32)
                  + jnp.where(lane == 14, inv, 0.0))


def _mid(parts, xr, w2t, wr2t, blg, scnt, s2, sm):
    return pl.pallas_call(
        _mid_body,
        grid=(GRID,),
        in_specs=[
            pl.BlockSpec((NC, 1, PBLK, D), lambda i: (0, i, 0, 0)),
            pl.BlockSpec((1, PBLK, D), lambda i: (i, 0, 0)),
            pl.BlockSpec((D, D), lambda i: (0, 0)),
            pl.BlockSpec((D, D), lambda i: (0, 0)),
            pl.BlockSpec((3, D), lambda i: (0, 0)),
            pl.BlockSpec((D, D), lambda i: (0, 0)),
            pl.BlockSpec((D, D), lambda i: (0, 0)),
            pl.BlockSpec((D, D), lambda i: (0, 0)),
        ],
        out_specs=[
            pl.BlockSpec((1, PBLK, D), lambda i: (i, 0, 0)),
            pl.BlockSpec((1, PBLK, D), lambda i: (i, 0, 0)),
        ],
        out_shape=[
            jax.ShapeDtypeStruct((GRID, PBLK, D), f32),
            jax.ShapeDtypeStruct((GRID, PBLK, D), f32),
        ],
    )(parts, xr, w2t, wr2t, blg, scnt, s2, sm)


NCH = NCOMP_PAD // 128
NSTR = 7


def _select_body(p2_hbm, xr2_hbm, idx_hbm, cid_hbm, bl2_hbm, out_hbm,
                 idx_v, cid_v, g_v, bl2_v, row_v, out_v, shr, sem, rsem):
    c = lax.axis_index("c")
    s = lax.axis_index("s")

    @pl.when((c == 0) & (s < NSTR))
    def _():
        pltpu.sync_copy(idx_hbm.at[s], idx_v.at[0])

        @pl.when(s < 4)
        def _():
            for j in range(NCH):
                pltpu.async_copy(p2_hbm.at[idx_v.at[0, j]], g_v.at[0, j], sem)

        @pl.when(s >= 4)
        def _():
            for j in range(NCH):
                pltpu.async_copy(xr2_hbm.at[idx_v.at[0, j]], g_v.at[0, j], sem)
        for j in range(NCH):
            pltpu.make_async_copy(p2_hbm.at[idx_v.at[0, j]], g_v.at[0, j], sem).wait()
        pltpu.sync_copy(g_v.at[0], shr.at[s])

    @pl.when(c == 0)
    def _():
        plsc.subcore_barrier()

    @pl.when((c == 0) & (s == 0))
    def _():
        pltpu.sync_copy(shr, g_v)
        pltpu.sync_copy(cid_hbm, cid_v)
        pltpu.sync_copy(bl2_hbm, bl2_v)
        bl2 = bl2_v[pl.ds(0, L)]
        bsum = bl2[0] + bl2[1]
        best_v = jnp.full((L,), -3e38, f32)
        best_n = jnp.zeros((L,), i32)
        best_p = jnp.full((L,), 2**30, i32)
        lane = lax.iota(i32, L)
        for j in range(NCH):
            for k in range(128 // L):
                sl = pl.ds(k * L, L)
                p0a = g_v.at[0, j][sl]
                p0b = g_v.at[1, j][sl]
                p1a = g_v.at[2, j][sl]
                p1b = g_v.at[3, j][sl]
                xa = g_v.at[4, j][sl]
                xb = g_v.at[5, j][sl]
                xinv = g_v.at[6, j][sl]
                v = (p0a + p1a + p0b + p1b) * xinv + xa + xb + bsum
                cid = cid_v.at[j][sl]
                pos = lane + (j * 128 + k * L)
                upd = (v > best_v) | ((v == best_v) & (pos < best_p))
                best_v = jnp.where(upd, v, best_v)
                best_n = jnp.where(upd, cid, best_n)
                best_p = jnp.where(upd, pos, best_p)
        bv, bn, bp = best_v[0], best_n[0], best_p[0]
        for l in range(1, L):
            v = best_v[l]
            take = (v > bv) | ((v == bv) & (best_p[l] < bp))
            bv = jnp.where(take, v, bv)
            bn = jnp.where(take, best_n[l], bn)
            bp = jnp.where(take, best_p[l], bp)
        pltpu.async_copy(p2_hbm.at[pl.ds(bn * L, L)], row_v.at[0], rsem)
        pltpu.async_copy(p2_hbm.at[pl.ds(N * L + bn * L, L)], row_v.at[1], rsem)
        pltpu.async_copy(xr2_hbm.at[pl.ds(bn * L, L)], row_v.at[2], rsem)
        for r in range(3):
            pltpu.make_async_copy(p2_hbm.at[pl.ds(0, L)], row_v.at[r], rsem).wait()
        xrow = row_v.at[2][pl.ds(0, L)]
        inv_s = jnp.full((L,), xrow[14], f32)
        rv = ((row_v.at[0][pl.ds(0, L)] + row_v.at[1][pl.ds(0, L)]) * inv_s
              + bl2 + xrow)
        pv = rv[2]
        pi = jnp.int32(0)
        for l in range(3, 14):
            v = rv[l]
            take = v > pv
            pv = jnp.where(take, v, pv)
            pi = jnp.where(take, jnp.int32(l - 2), pi)
        out_v[...] = jnp.where(lane == 0, bn, 0) + jnp.where(lane == 1, pi, 0)
        pltpu.sync_copy(out_v, out_hbm)


def _select(p2_flat, xr2_flat, idx, cid, bl2):
    k = pl.kernel(
        _select_body,
        out_type=jax.ShapeDtypeStruct((L,), i32),
        mesh=plsc.VectorSubcoreMesh(core_axis_name="c", subcore_axis_name="s"),
        scratch_types=[
            pltpu.VMEM((1, NCH, 128), i32),
            pltpu.VMEM((NCH, 128), i32),
            pltpu.VMEM((NSTR, NCH, 128), f32),
            pltpu.VMEM((L,), f32),
            pltpu.VMEM((3, L), f32),
            pltpu.VMEM((L,), i32),
            pltpu.VMEM_SHARED((NSTR, NCH, 128), f32),
            pltpu.SemaphoreType.DMA,
            pltpu.SemaphoreType.DMA,
        ],
    )
    return k(p2_flat, xr2_flat, idx, cid, bl2)


def kernel(x, edge_index, ts_beats, divs_pq, onset_div, duration_div,
           not_removed_notes, computation_notes, target,
           params_op, params_idx, params_pitch):
    del ts_beats, divs_pq, onset_div, duration_div, not_removed_notes
    del target, params_op
    pi, pp = params_idx, params_pitch

    pt = jnp.concatenate([pi['c1']['Wp'], pp['c1']['Wp']], axis=0).T
    bp = jnp.concatenate([pi['c1']['bp'], pp['c1']['bp']]).reshape(1, 2 * D)
    w1t = (jnp.zeros((2 * D, L), f32)
           .at[:D, 0:2].set(pi['c1']['Wl'].T)
           .at[D:, 2:14].set(pp['c1']['Wl'].T))
    wrt = (jnp.zeros((D, L), f32)
           .at[:, 0:2].set(pi['c1']['Wr'].T)
           .at[:, 2:14].set(pp['c1']['Wr'].T))
    blg = (jnp.zeros((3, L), f32)
           .at[0, 0:2].set(pi['c1']['bl']).at[0, 2:14].set(pp['c1']['bl'])
           .at[1, 0:2].set(pi['ln_g']).at[1, 2:14].set(pp['ln_g'])
           .at[2, 0:2].set(pi['ln_b']).at[2, 2:14].set(pp['ln_b']))
    blg = jnp.tile(blg, (1, 8))
    w2t = (jnp.zeros((L, L), f32)
           .at[0:2, 0:2].set(pi['c2']['Wl'].T)
           .at[2:14, 2:14].set(pp['c2']['Wl'].T))
    wr2t = (jnp.zeros((L, L), f32)
            .at[0:2, 0:2].set(pi['c2']['Wr'].T)
            .at[2:14, 2:14].set(pp['c2']['Wr'].T))
    bl2 = (jnp.zeros((L,), f32)
           .at[0:2].set(pi['c2']['bl']).at[2:14].set(pp['c2']['bl']))

    edges = edge_index.astype(i32).reshape(2, NCHK, CLEN)
    zer = jnp.zeros((N, L), f32)

    comp = computation_notes.astype(i32)
    comp_pad = jnp.concatenate([comp, jnp.broadcast_to(comp[0], (NCOMP_PAD - NCOMP,))])
    cid = comp_pad.reshape(NCH, 128)
    base = cid * L
    idx = jnp.stack([base, base + 1, N * L + base, N * L + base + 1,
                     base, base + 1, base + 14])

    scnt = jnp.asarray(_SCNT)
    s2 = jnp.asarray(_S2)
    sm = jnp.asarray(_SM)

    y1, xr = _proj(x, pt.astype(bf16), bp, w1t.astype(bf16), wrt.astype(bf16))
    parts1 = _edge_agg(y1.reshape(N, L), edges, zer)
    eye8 = jnp.asarray(_np.eye(8, dtype=_np.float32))
    w2b = jnp.kron(eye8, w2t)
    wr2b = jnp.kron(eye8, wr2t)
    y2, xr2 = _mid(parts1.reshape(NC, GRID, PBLK, D), xr.reshape(GRID, PBLK, D),
                   w2b, wr2b, blg, scnt, s2, sm)
    parts2 = _edge_agg(y2.reshape(N, L), edges, zer)
    sel = _select(parts2.reshape(NC * N * L), xr2.reshape(N * L), idx, cid, bl2)

    note_index = sel[0]
    new_pitch = sel[1]
    return x.at[note_index, :12].set(jax.nn.one_hot(new_pitch, 12, dtype=x.dtype))


# speedup vs baseline: 51.6330x; 1.0008x over previous
import jax
import jax.numpy as jnp
from jax import lax
from jax.experimental import pallas as pl
from jax.experimental.pallas import tpu as pltpu
from jax.experimental.pallas import tpu_sc as plsc

import numpy as _np

N = 10000
D = 128
E = 320000
L = 16
NC = 2
NS = 16
NW = NC * NS
CLEN = 128
NCHK = E // CLEN
CHUNKS = NCHK // NW
XTRA = NCHK - NW * CHUNKS
RPT = 624
REM = N - NS * RPT
RBLK = 2000
PBLK = RBLK // 8
GRID = N // RBLK
NP = N // 8
NCOMP = 1000
NCOMP_PAD = 1024

f32 = jnp.float32
i32 = jnp.int32
bf16 = jnp.bfloat16

_b = _np.zeros((L, L), _np.float32)
_b[14, :] = 1.0
_SCNT = _np.kron(_np.eye(8, dtype=_np.float32), _b)
_b2 = _np.zeros((L, L), _np.float32)
_b2[0:2, 0:2] = 1.0
_b2[2:14, 2:14] = 1.0
_S2 = _np.kron(_np.eye(8, dtype=_np.float32), _b2)
_bm = _b2 / _np.concatenate([_np.full(2, 2.0), _np.full(12, 12.0), _np.ones(2)]).astype(_np.float32)
_SM = _np.kron(_np.eye(8, dtype=_np.float32), _bm)


def _proj_body(x_ref, pt_ref, bp_ref, w1t_ref, wrt_ref, y1_ref, xr_ref):
    xb = x_ref[...].astype(bf16)
    p = jnp.maximum(jnp.dot(xb, pt_ref[...], preferred_element_type=f32) + bp_ref[...], 0.0)
    y1 = jnp.dot(p.astype(bf16), w1t_ref[...], preferred_element_type=f32)
    lane = lax.broadcasted_iota(i32, (RBLK, L), 1)
    y1_ref[...] = y1 + jnp.where(lane == 14, 1.0, 0.0)
    xr_ref[...] = jnp.dot(xb, wrt_ref[...], preferred_element_type=f32)


def _proj(x, pt, bp, w1t, wrt):
    return pl.pallas_call(
        _proj_body,
        grid=(GRID,),
        in_specs=[
            pl.BlockSpec((RBLK, D), lambda i: (i, 0)),
            pl.BlockSpec((D, 2 * D), lambda i: (0, 0)),
            pl.BlockSpec((1, 2 * D), lambda i: (0, 0)),
            pl.BlockSpec((2 * D, L), lambda i: (0, 0)),
            pl.BlockSpec((D, L), lambda i: (0, 0)),
        ],
        out_specs=[
            pl.BlockSpec((RBLK, L), lambda i: (i, 0)),
            pl.BlockSpec((RBLK, L), lambda i: (i, 0)),
        ],
        out_shape=[
            jax.ShapeDtypeStruct((N, L), f32),
            jax.ShapeDtypeStruct((N, L), f32),
        ],
    )(x, pt, bp, w1t, wrt)


NBUF = 12
GLEAD = NBUF // 2


def _edge_agg_body(y_hbm, edge_hbm, zer_hbm, out_hbm,
                   src_v, dst_v, srcx_v, dstx_v, rows, shared, gsem, ssem):
    c = lax.axis_index("c")
    s = lax.axis_index("s")
    w = c * NS + s
    pltpu.sync_copy(zer_hbm.at[pl.ds(s * RPT, RPT)], shared.at[pl.ds(s * RPT, RPT)])

    @pl.when(s == NS - 1)
    def _():
        pltpu.sync_copy(zer_hbm.at[pl.ds(NS * RPT, REM)], shared.at[pl.ds(NS * RPT, REM)])
    pltpu.sync_copy(edge_hbm.at[0, pl.ds(w * CHUNKS, CHUNKS)], src_v)
    pltpu.sync_copy(edge_hbm.at[1, pl.ds(w * CHUNKS, CHUNKS)], dst_v)

    @pl.when(w < XTRA)
    def _():
        pltpu.sync_copy(edge_hbm.at[0, pl.ds(NW * CHUNKS + w, 1)], srcx_v)
        pltpu.sync_copy(edge_hbm.at[1, pl.ds(NW * CHUNKS + w, 1)], dstx_v)
    plsc.subcore_barrier()

    for j in range(GLEAD):
        b = j % NBUF
        pltpu.async_copy(y_hbm.at[src_v.at[j]], rows.at[b], gsem.at[b])
    for j in range(CHUNKS):
        jg = j + GLEAD
        if jg < CHUNKS:
            bg = jg % NBUF
            if jg >= NBUF:
                pltpu.make_async_copy(
                    rows.at[bg], shared.at[dst_v.at[jg - NBUF]], ssem.at[bg]).wait()
            pltpu.async_copy(y_hbm.at[src_v.at[jg]], rows.at[bg], gsem.at[bg])
        b = j % NBUF
        pltpu.make_async_copy(y_hbm.at[src_v.at[j]], rows.at[b], gsem.at[b]).wait()
        pltpu.async_copy(rows.at[b], shared.at[dst_v.at[j]], ssem.at[b], add=True)
    for j in range(CHUNKS - NBUF, CHUNKS):
        b = j % NBUF
        pltpu.make_async_copy(
            rows.at[b], shared.at[dst_v.at[j]], ssem.at[b]).wait()

    @pl.when(w < XTRA)
    def _():
        pltpu.async_copy(y_hbm.at[srcx_v.at[0]], rows.at[0], gsem.at[0])
        pltpu.make_async_copy(y_hbm.at[srcx_v.at[0]], rows.at[0], gsem.at[0]).wait()
        pltpu.sync_copy(rows.at[0], shared.at[dstx_v.at[0]], add=True)
    plsc.subcore_barrier()
    pltpu.sync_copy(shared.at[pl.ds(s * RPT, RPT)], out_hbm.at[c, pl.ds(s * RPT, RPT)])

    @pl.when(s == NS - 1)
    def _():
        pltpu.sync_copy(shared.at[pl.ds(NS * RPT, REM)], out_hbm.at[c, pl.ds(NS * RPT, REM)])


def _edge_agg(y, edges, zer):
    k = pl.kernel(
        _edge_agg_body,
        out_type=jax.ShapeDtypeStruct((NC, N, L), f32),
        mesh=plsc.VectorSubcoreMesh(core_axis_name="c", subcore_axis_name="s"),
        scratch_types=[
            pltpu.VMEM((CHUNKS, CLEN), i32),
            pltpu.VMEM((CHUNKS, CLEN), i32),
            pltpu.VMEM((1, CLEN), i32),
            pltpu.VMEM((1, CLEN), i32),
            pltpu.VMEM((NBUF, CLEN, L), f32),
            pltpu.VMEM_SHARED((N, L), f32),
            pltpu.SemaphoreType.DMA((NBUF,)),
            pltpu.SemaphoreType.DMA((NBUF,)),
        ],
        compiler_params=pltpu.CompilerParams(use_tc_tiling_on_sc=False),
    )
    return k(y, edges, zer)


def _mid_body(parts_ref, xr_ref, w2t_ref, wr2t_ref, blg_ref,
              scnt_ref, s2_ref, sm_ref, y2_ref, xr2_ref):
    a = parts_ref[0, 0] + parts_ref[1, 0]
    blcat = blg_ref[0:1, :]
    gcat = blg_ref[1:2, :]
    bcat = blg_ref[2:3, :]
    lane = lax.broadcasted_iota(i32, (PBLK, D), 1) % L
    cnt = jnp.dot(jnp.where(lane == 14, a, 0.0), scnt_ref[...],
                  preferred_element_type=f32)
    inv = 1.0 / jnp.maximum(cnt, 1.0)
    o = jnp.where(lane < 14, a * inv + blcat + xr_ref[0], 0.0)
    nrm2 = jnp.dot(o * o, s2_ref[...], preferred_element_type=f32)
    o = o / jnp.maximum(jnp.sqrt(nrm2), 1e-12)
    o = jnp.maximum(o, 0.0)
    mean = jnp.dot(o, sm_ref[...], preferred_element_type=f32)
    dlt = o - mean
    var = jnp.dot(dlt * dlt, sm_ref[...], preferred_element_type=f32)
    h1 = jnp.where(lane < 14, dlt * lax.rsqrt(var + 1e-5) * gcat + bcat, 0.0)
    y2_ref[0] = jnp.dot(h1, w2t_ref[...], preferred_element_type=f32)
    xr2_ref[0] = (jnp.dot(h1, wr2t_ref[...], preferred_element_type=f32)
                  + jnp.where(lane == 14, inv, 0.0))


def _mid(parts, xr, w2t, wr2t, blg, scnt, s2, sm):
    return pl.pallas_call(
        _mid_body,
        grid=(GRID,),
        in_specs=[
            pl.BlockSpec((NC, 1, PBLK, D), lambda i: (0, i, 0, 0)),
            pl.BlockSpec((1, PBLK, D), lambda i: (i, 0, 0)),
            pl.BlockSpec((D, D), lambda i: (0, 0)),
            pl.BlockSpec((D, D), lambda i: (0, 0)),
            pl.BlockSpec((3, D), lambda i: (0, 0)),
            pl.BlockSpec((D, D), lambda i: (0, 0)),
            pl.BlockSpec((D, D), lambda i: (0, 0)),
            pl.BlockSpec((D, D), lambda i: (0, 0)),
        ],
        out_specs=[
            pl.BlockSpec((1, PBLK, D), lambda i: (i, 0, 0)),
            pl.BlockSpec((1, PBLK, D), lambda i: (i, 0, 0)),
        ],
        out_shape=[
            jax.ShapeDtypeStruct((GRID, PBLK, D), f32),
            jax.ShapeDtypeStruct((GRID, PBLK, D), f32),
        ],
    )(parts, xr, w2t, wr2t, blg, scnt, s2, sm)


NCH = NCOMP_PAD // 128
NSTR = 7


def _select_body(p2_hbm, xr2_hbm, idx_hbm, cid_hbm, bl2_hbm, out_hbm,
                 idx_v, cid_v, g_v, bl2_v, row_v, out_v, shr, sem, rsem):
    c = lax.axis_index("c")
    s = lax.axis_index("s")

    @pl.when((c == 0) & (s < NSTR))
    def _():
        pltpu.sync_copy(idx_hbm.at[s], idx_v.at[0])

        @pl.when(s < 4)
        def _():
            for j in range(NCH):
                pltpu.async_copy(p2_hbm.at[idx_v.at[0, j]], g_v.at[0, j], sem)

        @pl.when(s >= 4)
        def _():
            for j in range(NCH):
                pltpu.async_copy(xr2_hbm.at[idx_v.at[0, j]], g_v.at[0, j], sem)
        for j in range(NCH):
            pltpu.make_async_copy(p2_hbm.at[idx_v.at[0, j]], g_v.at[0, j], sem).wait()
        pltpu.sync_copy(g_v.at[0], shr.at[s])

    @pl.when(c == 0)
    def _():
        plsc.subcore_barrier()

    @pl.when((c == 0) & (s == 0))
    def _():
        pltpu.sync_copy(shr, g_v)
        pltpu.sync_copy(cid_hbm, cid_v)
        pltpu.sync_copy(bl2_hbm, bl2_v)
        bl2 = bl2_v[pl.ds(0, L)]
        bsum = bl2[0] + bl2[1]
        best_v = jnp.full((L,), -3e38, f32)
        best_n = jnp.zeros((L,), i32)
        best_p = jnp.full((L,), 2**30, i32)
        lane = lax.iota(i32, L)
        for j in range(NCH):
            for k in range(128 // L):
                sl = pl.ds(k * L, L)
                p0a = g_v.at[0, j][sl]
                p0b = g_v.at[1, j][sl]
                p1a = g_v.at[2, j][sl]
                p1b = g_v.at[3, j][sl]
                xa = g_v.at[4, j][sl]
                xb = g_v.at[5, j][sl]
                xinv = g_v.at[6, j][sl]
                v = (p0a + p1a + p0b + p1b) * xinv + xa + xb + bsum
                cid = cid_v.at[j][sl]
                pos = lane + (j * 128 + k * L)
                upd = (v > best_v) | ((v == best_v) & (pos < best_p))
                best_v = jnp.where(upd, v, best_v)
                best_n = jnp.where(upd, cid, best_n)
                best_p = jnp.where(upd, pos, best_p)
        bv, bn, bp = best_v[0], best_n[0], best_p[0]
        for l in range(1, L):
            v = best_v[l]
            take = (v > bv) | ((v == bv) & (best_p[l] < bp))
            bv = jnp.where(take, v, bv)
            bn = jnp.where(take, best_n[l], bn)
            bp = jnp.where(take, best_p[l], bp)
        pltpu.async_copy(p2_hbm.at[pl.ds(bn * L, L)], row_v.at[0], rsem)
        pltpu.async_copy(p2_hbm.at[pl.ds(N * L + bn * L, L)], row_v.at[1], rsem)
        pltpu.async_copy(xr2_hbm.at[pl.ds(bn * L, L)], row_v.at[2], rsem)
        for r in range(3):
            pltpu.make_async_copy(p2_hbm.at[pl.ds(0, L)], row_v.at[r], rsem).wait()
        xrow = row_v.at[2][pl.ds(0, L)]
        inv_s = jnp.full((L,), xrow[14], f32)
        rv = ((row_v.at[0][pl.ds(0, L)] + row_v.at[1][pl.ds(0, L)]) * inv_s
              + bl2 + xrow)
        pv = rv[2]
        pi = jnp.int32(0)
        for l in range(3, 14):
            v = rv[l]
            take = v > pv
            pv = jnp.where(take, v, pv)
            pi = jnp.where(take, jnp.int32(l - 2), pi)
        out_v[...] = jnp.where(lane == 0, bn, 0) + jnp.where(lane == 1, pi, 0)
        pltpu.sync_copy(out_v, out_hbm)


def _select(p2_flat, xr2_flat, idx, cid, bl2):
    k = pl.kernel(
        _select_body,
        out_type=jax.ShapeDtypeStruct((L,), i32),
        mesh=plsc.VectorSubcoreMesh(core_axis_name="c", subcore_axis_name="s"),
        scratch_types=[
            pltpu.VMEM((1, NCH, 128), i32),
            pltpu.VMEM((NCH, 128), i32),
            pltpu.VMEM((NSTR, NCH, 128), f32),
            pltpu.VMEM((L,), f32),
            pltpu.VMEM((3, L), f32),
            pltpu.VMEM((L,), i32),
            pltpu.VMEM_SHARED((NSTR, NCH, 128), f32),
            pltpu.SemaphoreType.DMA,
            pltpu.SemaphoreType.DMA,
        ],
    )
    return k(p2_flat, xr2_flat, idx, cid, bl2)


def kernel(x, edge_index, ts_beats, divs_pq, onset_div, duration_div,
           not_removed_notes, computation_notes, target,
           params_op, params_idx, params_pitch):
    del ts_beats, divs_pq, onset_div, duration_div, not_removed_notes
    del target, params_op
    pi, pp = params_idx, params_pitch

    pt = jnp.concatenate([pi['c1']['Wp'], pp['c1']['Wp']], axis=0).T
    bp = jnp.concatenate([pi['c1']['bp'], pp['c1']['bp']]).reshape(1, 2 * D)
    w1t = (jnp.zeros((2 * D, L), f32)
           .at[:D, 0:2].set(pi['c1']['Wl'].T)
           .at[D:, 2:14].set(pp['c1']['Wl'].T))
    wrt = (jnp.zeros((D, L), f32)
           .at[:, 0:2].set(pi['c1']['Wr'].T)
           .at[:, 2:14].set(pp['c1']['Wr'].T))
    blg = (jnp.zeros((3, L), f32)
           .at[0, 0:2].set(pi['c1']['bl']).at[0, 2:14].set(pp['c1']['bl'])
           .at[1, 0:2].set(pi['ln_g']).at[1, 2:14].set(pp['ln_g'])
           .at[2, 0:2].set(pi['ln_b']).at[2, 2:14].set(pp['ln_b']))
    blg = jnp.tile(blg, (1, 8))
    w2t = (jnp.zeros((L, L), f32)
           .at[0:2, 0:2].set(pi['c2']['Wl'].T)
           .at[2:14, 2:14].set(pp['c2']['Wl'].T))
    wr2t = (jnp.zeros((L, L), f32)
            .at[0:2, 0:2].set(pi['c2']['Wr'].T)
            .at[2:14, 2:14].set(pp['c2']['Wr'].T))
    bl2 = (jnp.zeros((L,), f32)
           .at[0:2].set(pi['c2']['bl']).at[2:14].set(pp['c2']['bl']))

    edges = edge_index.astype(i32).reshape(2, NCHK, CLEN)
    zer = jnp.zeros((N, L), f32)

    comp = computation_notes.astype(i32)
    comp_pad = jnp.concatenate([comp, jnp.broadcast_to(comp[0], (NCOMP_PAD - NCOMP,))])
    cid = comp_pad.reshape(NCH, 128)
    base = cid * L
    idx = jnp.stack([base, base + 1, N * L + base, N * L + base + 1,
                     base, base + 1, base + 14])

    scnt = jnp.asarray(_SCNT)
    s2 = jnp.asarray(_S2)
    sm = jnp.asarray(_SM)

    y1, xr = _proj(x, pt.astype(bf16), bp, w1t.astype(bf16), wrt.astype(bf16))
    parts1 = _edge_agg(y1.reshape(N, L), edges, zer)
    eye8 = jnp.asarray(_np.eye(8, dtype=_np.float32))
    w2b = jnp.kron(eye8, w2t)
    wr2b = jnp.kron(eye8, wr2t)
    y2, xr2 = _mid(parts1.reshape(NC, GRID, PBLK, D), xr.reshape(GRID, PBLK, D),
                   w2b, wr2b, blg, scnt, s2, sm)
    parts2 = _edge_agg(y2.reshape(N, L), edges, zer)
    sel = _select(parts2.reshape(NC * N * L), xr2.reshape(N * L), idx, cid, bl2)

    note_index = sel[0]
    new_pitch = sel[1]
    return x.at[note_index, :12].set(jax.nn.one_hot(new_pitch, 12, dtype=x.dtype))
